# baseline (speedup 1.0000x reference)
import functools

import jax
import jax.numpy as jnp
import numpy as np
from jax import lax
from jax.experimental import pallas as pl
from jax.experimental.pallas import tpu as pltpu
from jax.experimental.pallas import tpu_sc as plsc

N = 10000
D = 128
E = 320000
T = 80000
H1, C1, F1 = 8, 8, 64
F2 = 64

NC, NS = 2, 16
NW = NC * NS
R = 80

EPT = E // NW
KE = 80
QE = EPT // KE

KD = 128
QD_TOT = T // KD
QD = 20
TPAD = NW * QD * KD


def _mesh():
    return plsc.VectorSubcoreMesh(
        core_axis_name="c", subcore_axis_name="s",
        num_cores=NC, num_subcores=NS)


def _leaky(x):
    return jnp.maximum(x, 0.2 * x)


def _tca_body(x_ref, w1t_ref, p1_ref, ad1m_ref, rec1_ref, ad1t_ref):
    xp = jnp.dot(x_ref[...], w1t_ref[...], preferred_element_type=jnp.float32)
    rec1_ref[...] = jnp.dot(xp, p1_ref[...], preferred_element_type=jnp.float32)
    ad1t_ref[...] = jnp.dot(xp, ad1m_ref[...],
                            preferred_element_type=jnp.float32)


def _tc_stage_a(x, w1t, p1, ad1m):
    blk = 1000
    return pl.pallas_call(
        _tca_body,
        grid=(N // blk,),
        in_specs=[
            pl.BlockSpec((blk, D), lambda i: (i, 0)),
            pl.BlockSpec((D, F1), lambda i: (0, 0)),
            pl.BlockSpec((F1, R), lambda i: (0, 0)),
            pl.BlockSpec((F1, 16), lambda i: (0, 0)),
        ],
        out_specs=[
            pl.BlockSpec((blk, R), lambda i: (i, 0)),
            pl.BlockSpec((blk, 16), lambda i: (i, 0)),
        ],
        out_shape=[
            jax.ShapeDtypeStruct((N, R), jnp.float32),
            jax.ShapeDtypeStruct((N, 16), jnp.float32),
        ],
    )(x, w1t, p1, ad1m)


def _tcb_body(a0_ref, a1_ref, rec1_ref, ad1t_ref, w2t_ref,
              p2_ref, ad2m_ref, b8il_ref, pm_ref, b1_ref, rec2_ref, ad2t_ref):
    xp_il = rec1_ref[:, 0:F1]
    asrc = rec1_ref[:, 64:72]
    adst = ad1t_ref[:, 0:8]
    wself = jnp.exp(_leaky(asrc + adst))
    b8il = b8il_ref[...]
    numer = (a0_ref[:, 0:F1] + a1_ref[:, 0:F1]
             + xp_il * jnp.dot(wself, b8il, preferred_element_type=jnp.float32))
    den8 = a0_ref[:, 64:72] + a1_ref[:, 64:72] + wself
    z_il = numer / jnp.dot(den8, b8il, preferred_element_type=jnp.float32)
    z = jnp.dot(z_il, pm_ref[...], preferred_element_type=jnp.float32) \
        + b1_ref[...]
    z = jnp.where(z > 0, z, jnp.exp(z) - 1.0)
    xp2 = jnp.dot(z, w2t_ref[...], preferred_element_type=jnp.float32)
    rec2_ref[...] = jnp.dot(xp2, p2_ref[...],
                            preferred_element_type=jnp.float32)
    ad2t_ref[...] = jnp.dot(xp2, ad2m_ref[...],
                            preferred_element_type=jnp.float32)


def _tc_stage_b(a0, a1, rec1, ad1t, w2t, p2, ad2m, b8il, pm, b1row):
    blk = 1000
    return pl.pallas_call(
        _tcb_body,
        grid=(N // blk,),
        in_specs=[
            pl.BlockSpec((blk, R), lambda i: (i, 0)),
            pl.BlockSpec((blk, R), lambda i: (i, 0)),
            pl.BlockSpec((blk, R), lambda i: (i, 0)),
            pl.BlockSpec((blk, 16), lambda i: (i, 0)),
            pl.BlockSpec((F1, F2), lambda i: (0, 0)),
            pl.BlockSpec((F2, R), lambda i: (0, 0)),
            pl.BlockSpec((F2, 16), lambda i: (0, 0)),
            pl.BlockSpec((8, F1), lambda i: (0, 0)),
            pl.BlockSpec((F1, F1), lambda i: (0, 0)),
            pl.BlockSpec((1, F1), lambda i: (0, 0)),
        ],
        out_specs=[
            pl.BlockSpec((blk, R), lambda i: (i, 0)),
            pl.BlockSpec((blk, 16), lambda i: (i, 0)),
        ],
        out_shape=[
            jax.ShapeDtypeStruct((N, R), jnp.float32),
            jax.ShapeDtypeStruct((N, 16), jnp.float32),
        ],
    )(a0, a1, rec1, ad1t, w2t, p2, ad2m, b8il, pm, b1row)


def _tcc_body(b0_ref, b1p_ref, rec2_ref, ad2t_ref,
              ones64_ref, b2_ref, z2_ref):
    xp2 = rec2_ref[:, 0:F2]
    as2 = rec2_ref[:, 64:65]
    ad2 = ad2t_ref[:, 0:1]
    w2 = jnp.exp(_leaky(as2 + ad2))
    ones64 = ones64_ref[...]
    numer = (b0_ref[:, 0:F2] + b1p_ref[:, 0:F2]
             + xp2 * jnp.dot(w2, ones64, preferred_element_type=jnp.float32))
    denom = jnp.dot(b0_ref[:, 64:65] + b1p_ref[:, 64:65] + w2, ones64,
                    preferred_element_type=jnp.float32)
    z2_ref[...] = numer / denom + b2_ref[...]


def _tc_stage_c(b0, b1p, rec2, ad2t, ones64, b2row):
    blk = 1000
    return pl.pallas_call(
        _tcc_body,
        grid=(N // blk,),
        in_specs=[
            pl.BlockSpec((blk, R), lambda i: (i, 0)),
            pl.BlockSpec((blk, R), lambda i: (i, 0)),
            pl.BlockSpec((blk, R), lambda i: (i, 0)),
            pl.BlockSpec((blk, 16), lambda i: (i, 0)),
            pl.BlockSpec((1, F2), lambda i: (0, 0)),
            pl.BlockSpec((1, F2), lambda i: (0, 0)),
        ],
        out_specs=[pl.BlockSpec((blk, F2), lambda i: (i, 0))],
        out_shape=[jax.ShapeDtypeStruct((N, F2), jnp.float32)],
    )(b0, b1p, rec2, ad2t, ones64, b2row)


def _iota16():
    return lax.iota(jnp.int32, 16)


def _lane_gather(v, idx):
    return lax.gather(
        v, idx[:, None],
        dimension_numbers=lax.GatherDimensionNumbers(
            offset_dims=(), collapsed_slice_dims=(0,), start_index_map=(0,)),
        slice_sizes=(1,),
        mode=lax.GatherScatterMode.PROMISE_IN_BOUNDS)


def _hsum_all(v):
    iot = _iota16()
    for off in (8, 4, 2, 1):
        v = v + _lane_gather(v, iot ^ off)
    return v


@functools.lru_cache(maxsize=None)
def _sc_edge_kernel(heads):
    @functools.partial(
        pl.kernel,
        out_type=[
            jax.ShapeDtypeStruct((N, R), jnp.float32),
            jax.ShapeDtypeStruct((N, R), jnp.float32),
        ],
        mesh=_mesh(),
        compiler_params=pltpu.CompilerParams(use_tc_tiling_on_sc=False),
        scratch_types=[
            pltpu.VMEM((QE, KE), jnp.int32),
            pltpu.VMEM((QE, KE), jnp.int32),
            pltpu.VMEM((KE, R), jnp.float32),
            pltpu.VMEM((KE, R), jnp.float32),
            pltpu.VMEM((KE, 16), jnp.float32),
            pltpu.VMEM((KE, 16), jnp.float32),
            pltpu.VMEM((KE, R), jnp.float32),
            pltpu.VMEM((KE, R), jnp.float32),
            pltpu.VMEM_SHARED((N, R), jnp.float32),
            pltpu.SemaphoreType.DMA,
            pltpu.SemaphoreType.DMA,
            pltpu.SemaphoreType.DMA,
            pltpu.SemaphoreType.DMA,
        ],
    )
    def sc_edges(src2d_hbm, dst2d_hbm, rec_hbm, adt_hbm, zr_hbm,
                 accf0_hbm, accf1_hbm,
                 sbig, dbig, recb0, recb1, adb0, adb1, mb0, mb1,
                 accf_sp, g0, g1, s0, s1):
        c = lax.axis_index("c")
        s = lax.axis_index("s")
        tid = s * NC + c

        recb = (recb0, recb1)
        adb = (adb0, adb1)
        mb = (mb0, mb1)
        gsem = (g0, g1)
        ssem = (s0, s1)

        pltpu.sync_copy(src2d_hbm.at[pl.ds(tid * QE, QE)], sbig)
        pltpu.sync_copy(dst2d_hbm.at[pl.ds(tid * QE, QE)], dbig)

        rpt = 624
        tail0 = rpt * NS
        r0 = s * rpt
        pltpu.sync_copy(zr_hbm.at[pl.ds(r0, rpt)], accf_sp.at[pl.ds(r0, rpt)])

        @pl.when(s == NS - 1)
        def _():
            pltpu.sync_copy(zr_hbm.at[pl.ds(tail0, N - tail0)],
                            accf_sp.at[pl.ds(tail0, N - tail0)])

        plsc.subcore_barrier()

        iot = _iota16()
        pat8 = iot & 7

        def issue_gathers(q, b):
            pltpu.async_copy(rec_hbm.at[sbig.at[q]], recb[b], gsem[b])
            pltpu.async_copy(adt_hbm.at[dbig.at[q]], adb[b], gsem[b])

        def wait_gathers(b):
            pltpu.make_async_copy(rec_hbm.at[sbig.at[0]], recb[b],
                                  gsem[b]).wait()
            pltpu.make_async_copy(adt_hbm.at[dbig.at[0]], adb[b],
                                  gsem[b]).wait()

        def issue_scatters(q, b):
            pltpu.async_copy(mb[b], accf_sp.at[dbig.at[q]], ssem[b], add=True)

        def wait_scatters(b):
            pltpu.make_async_copy(mb[b], accf_sp.at[dbig.at[0]],
                                  ssem[b]).wait()

        def compute(b):
            rb = recb[b]
            ab = adb[b]
            mbb = mb[b]

            def edge_body(k, carry):
                alpha = rb[k, pl.ds(64, 16)] + ab[k, :]
                w = jnp.exp(_leaky(alpha))
                mbb[k, pl.ds(64, 16)] = w
                wv = _lane_gather(w, pat8) if heads == 8 else w
                for j in range(4):
                    mbb[k, pl.ds(16 * j, 16)] = rb[k, pl.ds(16 * j, 16)] * wv
                return carry

            lax.fori_loop(0, KE, edge_body, 0, unroll=4)

        def half(q, b):
            @pl.when(q >= 1)
            def _():
                wait_scatters(1 - b)

            @pl.when(q + 1 < QE)
            def _():
                issue_gathers(q + 1, 1 - b)

            wait_gathers(b)
            compute(b)
            issue_scatters(q, b)

        issue_gathers(0, 0)

        def pair_body(p, carry):
            half(2 * p, 0)
            half(2 * p + 1, 1)
            return carry

        lax.fori_loop(0, QE // 2, pair_body, 0)
        half(QE - 1, 0)
        wait_scatters(0)

        plsc.subcore_barrier()

        def dump(sp_ref, hbm_ref):
            pltpu.sync_copy(sp_ref.at[pl.ds(r0, rpt)],
                            hbm_ref.at[pl.ds(r0, rpt)])

            @pl.when(s == NS - 1)
            def _():
                pltpu.sync_copy(sp_ref.at[pl.ds(tail0, N - tail0)],
                                hbm_ref.at[pl.ds(tail0, N - tail0)])

        @pl.when(c == 0)
        def _():
            dump(accf_sp, accf0_hbm)

        @pl.when(c == 1)
        def _():
            dump(accf_sp, accf1_hbm)

    return sc_edges


@functools.lru_cache(maxsize=None)
def _sc_decode_kernel():
    @functools.partial(
        pl.kernel,
        out_type=jax.ShapeDtypeStruct((T,), jnp.float32),
        mesh=_mesh(),
        compiler_params=pltpu.CompilerParams(use_tc_tiling_on_sc=False),
        scratch_types=[
            pltpu.VMEM((QD, KD), jnp.int32),
            pltpu.VMEM((QD, KD), jnp.int32),
            pltpu.VMEM((KD, F2), jnp.float32),
            pltpu.VMEM((KD, F2), jnp.float32),
            pltpu.VMEM((KD, F2), jnp.float32),
            pltpu.VMEM((KD, F2), jnp.float32),
            pltpu.VMEM((KD,), jnp.float32),
            pltpu.VMEM((KD,), jnp.float32),
            pltpu.SemaphoreType.DMA,
            pltpu.SemaphoreType.DMA,
            pltpu.SemaphoreType.DMA,
            pltpu.SemaphoreType.DMA,
        ],
    )
    def sc_decode(z2_hbm, i0p_hbm, i1p_hbm, out_hbm,
                  ibig0, ibig1, za0, za1, zb0, zb1, ob0, ob1,
                  g0, g1, s0, s1):
        c = lax.axis_index("c")
        s = lax.axis_index("s")
        tid = s * NC + c

        za = (za0, za1)
        zb = (zb0, zb1)
        ob = (ob0, ob1)
        gsem = (g0, g1)
        ssem = (s0, s1)

        base_cnt = QD_TOT // NW
        rem = QD_TOT % NW
        cnt = base_cnt + jnp.where(tid < rem, 1, 0)
        start = base_cnt * tid + jnp.minimum(tid, rem)

        pltpu.sync_copy(i0p_hbm.at[pl.ds(start, QD)], ibig0)
        pltpu.sync_copy(i1p_hbm.at[pl.ds(start, QD)], ibig1)

        iot = _iota16()

        def issue_gathers(q, b):
            pltpu.async_copy(z2_hbm.at[ibig0.at[q]], za[b], gsem[b])
            pltpu.async_copy(z2_hbm.at[ibig1.at[q]], zb[b], gsem[b])

        def wait_gathers(b):
            pltpu.make_async_copy(z2_hbm.at[ibig0.at[0]], za[b],
                                  gsem[b]).wait()
            pltpu.make_async_copy(z2_hbm.at[ibig1.at[0]], zb[b],
                                  gsem[b]).wait()

        def compute(b):
            zab = za[b]
            zbb = zb[b]
            obb = ob[b]

            def group_body(g2, carry):
                def edge_body(e, gacc):
                    k = g2 * 16 + e
                    acc = zab[k, pl.ds(0, 16)] * zbb[k, pl.ds(0, 16)]
                    for j in range(1, 4):
                        acc = acc + (zab[k, pl.ds(16 * j, 16)]
                                     * zbb[k, pl.ds(16 * j, 16)])
                    return jnp.where(iot == e, _hsum_all(acc), gacc)

                gacc = lax.fori_loop(0, 16, edge_body,
                                     jnp.zeros((16,), jnp.float32), unroll=4)
                obb[pl.ds(g2 * 16, 16)] = gacc
                return carry

            lax.fori_loop(0, KD // 16, group_body, 0)

        def half(q, b):
            @pl.when(q + 1 < cnt)
            def _():
                issue_gathers(q + 1, 1 - b)

            wait_gathers(b)

            @pl.when(q >= 2)
            def _():
                pltpu.make_async_copy(
                    ob[b], out_hbm.at[pl.ds(0, KD)], ssem[b]).wait()

            compute(b)
            pltpu.async_copy(ob[b], out_hbm.at[pl.ds((start + q) * KD, KD)],
                             ssem[b])

        issue_gathers(0, 0)

        def pair_body(p, carry):
            @pl.when(2 * p < cnt)
            def _():
                half(2 * p, 0)

            @pl.when(2 * p + 1 < cnt)
            def _():
                half(2 * p + 1, 1)

            return carry

        lax.fori_loop(0, QD // 2, pair_body, 0)

        pltpu.make_async_copy(ob[0], out_hbm.at[pl.ds(0, KD)], ssem[0]).wait()
        pltpu.make_async_copy(ob[1], out_hbm.at[pl.ds(0, KD)], ssem[1]).wait()

    return sc_decode


def kernel(x, train_pos_edge_index, test_pos_edge_index, test_neg_edge_index,
           W1, a_src1, a_dst1, b1, W2, a_src2, a_dst2, b2):
    f32 = jnp.float32
    src2d = train_pos_edge_index[0].reshape(E // KE, KE)
    dst2d = train_pos_edge_index[1].reshape(E // KE, KE)
    ei = jnp.concatenate([test_pos_edge_index, test_neg_edge_index], axis=-1)
    npad = QD_TOT * KD // KD
    i0p = jnp.zeros((TPAD // KD, KD), ei.dtype).at[:npad].set(
        ei[0].reshape(npad, KD))
    i1p = jnp.zeros((TPAD // KD, KD), ei.dtype).at[:npad].set(
        ei[1].reshape(npad, KD))

    cols = np.arange(F1)
    perm = (cols % 8) * 8 + cols // 8
    pp = jnp.zeros((F1, F1), f32).at[perm, cols].set(1.0)
    pm = jnp.zeros((F1, F1), f32).at[cols, perm].set(1.0)
    hh = jnp.repeat(jnp.arange(H1), C1)
    asrc_bd = jnp.zeros((F1, 8), f32).at[jnp.arange(F1), hh].set(
        a_src1.reshape(F1))
    adst_bd = jnp.zeros((F1, 8), f32).at[jnp.arange(F1), hh].set(
        a_dst1.reshape(F1))
    p1 = jnp.concatenate([pp, asrc_bd, jnp.zeros((F1, 8), f32)], axis=1)
    ad1m = jnp.concatenate([adst_bd, jnp.zeros((F1, 8), f32)], axis=1)
    b8il = jnp.zeros((8, F1), f32).at[cols % 8, cols].set(1.0)

    p2 = jnp.concatenate(
        [jnp.eye(F2, dtype=f32),
         jnp.outer(a_src2.reshape(F2), jnp.ones((16,), f32))], axis=1)
    ad2m = jnp.outer(a_dst2.reshape(F2), jnp.ones((16,), f32))
    ones64 = jnp.ones((1, F2), f32)

    zr = jnp.zeros((N, R), f32)

    rec1, ad1t = _tc_stage_a(x, W1.T, p1, ad1m)
    a0, a1 = _sc_edge_kernel(8)(src2d, dst2d, rec1, ad1t, zr)
    rec2, ad2t = _tc_stage_b(a0, a1, rec1, ad1t, W2.T, p2, ad2m,
                             b8il, pm, b1.reshape(1, F1))
    b0, b1p = _sc_edge_kernel(1)(src2d, dst2d, rec2, ad2t, zr)
    (z2,) = _tc_stage_c(b0, b1p, rec2, ad2t, ones64,
                        b2.reshape(1, F2))
    logits = _sc_decode_kernel()(z2, i0p, i1p)
    return logits


# speedup vs baseline: 1.3011x; 1.3011x over previous
import functools

import jax
import jax.numpy as jnp
import numpy as np
from jax import lax
from jax.experimental import pallas as pl
from jax.experimental.pallas import tpu as pltpu
from jax.experimental.pallas import tpu_sc as plsc

N = 10000
D = 128
E = 320000
T = 80000
H1, C1, F1 = 8, 8, 64
F2 = 64

NC, NS = 2, 16
NW = NC * NS
R = 80

EPT = E // NW
KE = 80
QE = EPT // KE

KD = 128
QD_TOT = T // KD
QD = 20
TPAD = NW * QD * KD


def _mesh():
    return plsc.VectorSubcoreMesh(
        core_axis_name="c", subcore_axis_name="s",
        num_cores=NC, num_subcores=NS)


def _leaky(x):
    return jnp.maximum(x, 0.2 * x)


def _tca_body(x_ref, w1t_ref, p1_ref, ad1m_ref, rec1_ref, ad1t_ref):
    xp = jnp.dot(x_ref[...], w1t_ref[...], preferred_element_type=jnp.float32)
    rec1_ref[...] = jnp.dot(xp, p1_ref[...], preferred_element_type=jnp.float32)
    ad1t_ref[...] = jnp.dot(xp, ad1m_ref[...],
                            preferred_element_type=jnp.float32)


def _tc_stage_a(x, w1t, p1, ad1m):
    blk = 1000
    return pl.pallas_call(
        _tca_body,
        grid=(N // blk,),
        in_specs=[
            pl.BlockSpec((blk, D), lambda i: (i, 0)),
            pl.BlockSpec((D, F1), lambda i: (0, 0)),
            pl.BlockSpec((F1, R), lambda i: (0, 0)),
            pl.BlockSpec((F1, 16), lambda i: (0, 0)),
        ],
        out_specs=[
            pl.BlockSpec((blk, R), lambda i: (i, 0)),
            pl.BlockSpec((blk, 16), lambda i: (i, 0)),
        ],
        out_shape=[
            jax.ShapeDtypeStruct((N, R), jnp.float32),
            jax.ShapeDtypeStruct((N, 16), jnp.float32),
        ],
    )(x, w1t, p1, ad1m)


def _tcb_body(a0_ref, a1_ref, a0w_ref, a1w_ref, rec1_ref, ad1t_ref, w2t_ref,
              p2_ref, ad2m_ref, b8il_ref, pm_ref, b1_ref, rec2_ref, ad2t_ref):
    xp_il = rec1_ref[:, 0:F1]
    asrc = rec1_ref[:, 64:72]
    adst = ad1t_ref[:, 0:8]
    wself = jnp.exp(_leaky(asrc + adst))
    b8il = b8il_ref[...]
    numer = (a0_ref[:, 0:F1] + a1_ref[:, 0:F1]
             + xp_il * jnp.dot(wself, b8il, preferred_element_type=jnp.float32))
    den8 = a0w_ref[:, 0:8] + a1w_ref[:, 0:8] + wself
    z_il = numer / jnp.dot(den8, b8il, preferred_element_type=jnp.float32)
    z = jnp.dot(z_il, pm_ref[...], preferred_element_type=jnp.float32) \
        + b1_ref[...]
    z = jnp.where(z > 0, z, jnp.exp(z) - 1.0)
    xp2 = jnp.dot(z, w2t_ref[...], preferred_element_type=jnp.float32)
    rec2_ref[...] = jnp.dot(xp2, p2_ref[...],
                            preferred_element_type=jnp.float32)
    ad2t_ref[...] = jnp.dot(xp2, ad2m_ref[...],
                            preferred_element_type=jnp.float32)


def _tc_stage_b(a0, a1, a0w, a1w, rec1, ad1t, w2t, p2, ad2m, b8il, pm, b1row):
    blk = 1000
    return pl.pallas_call(
        _tcb_body,
        grid=(N // blk,),
        in_specs=[
            pl.BlockSpec((blk, R), lambda i: (i, 0)),
            pl.BlockSpec((blk, R), lambda i: (i, 0)),
            pl.BlockSpec((blk, 16), lambda i: (i, 0)),
            pl.BlockSpec((blk, 16), lambda i: (i, 0)),
            pl.BlockSpec((blk, R), lambda i: (i, 0)),
            pl.BlockSpec((blk, 16), lambda i: (i, 0)),
            pl.BlockSpec((F1, F2), lambda i: (0, 0)),
            pl.BlockSpec((F2, R), lambda i: (0, 0)),
            pl.BlockSpec((F2, 16), lambda i: (0, 0)),
            pl.BlockSpec((8, F1), lambda i: (0, 0)),
            pl.BlockSpec((F1, F1), lambda i: (0, 0)),
            pl.BlockSpec((1, F1), lambda i: (0, 0)),
        ],
        out_specs=[
            pl.BlockSpec((blk, R), lambda i: (i, 0)),
            pl.BlockSpec((blk, 16), lambda i: (i, 0)),
        ],
        out_shape=[
            jax.ShapeDtypeStruct((N, R), jnp.float32),
            jax.ShapeDtypeStruct((N, 16), jnp.float32),
        ],
    )(a0, a1, a0w, a1w, rec1, ad1t, w2t, p2, ad2m, b8il, pm, b1row)


def _tcc_body(b0_ref, b1p_ref, b0w_ref, b1w_ref, rec2_ref, ad2t_ref,
              ones64_ref, b2_ref, z2_ref):
    xp2 = rec2_ref[:, 0:F2]
    as2 = rec2_ref[:, 64:65]
    ad2 = ad2t_ref[:, 0:1]
    w2 = jnp.exp(_leaky(as2 + ad2))
    ones64 = ones64_ref[...]
    numer = (b0_ref[:, 0:F2] + b1p_ref[:, 0:F2]
             + xp2 * jnp.dot(w2, ones64, preferred_element_type=jnp.float32))
    denom = jnp.dot(b0w_ref[:, 0:1] + b1w_ref[:, 0:1] + w2, ones64,
                    preferred_element_type=jnp.float32)
    z2_ref[...] = numer / denom + b2_ref[...]


def _tc_stage_c(b0, b1p, b0w, b1w, rec2, ad2t, ones64, b2row):
    blk = 1000
    return pl.pallas_call(
        _tcc_body,
        grid=(N // blk,),
        in_specs=[
            pl.BlockSpec((blk, R), lambda i: (i, 0)),
            pl.BlockSpec((blk, R), lambda i: (i, 0)),
            pl.BlockSpec((blk, 16), lambda i: (i, 0)),
            pl.BlockSpec((blk, 16), lambda i: (i, 0)),
            pl.BlockSpec((blk, R), lambda i: (i, 0)),
            pl.BlockSpec((blk, 16), lambda i: (i, 0)),
            pl.BlockSpec((1, F2), lambda i: (0, 0)),
            pl.BlockSpec((1, F2), lambda i: (0, 0)),
        ],
        out_specs=[pl.BlockSpec((blk, F2), lambda i: (i, 0))],
        out_shape=[jax.ShapeDtypeStruct((N, F2), jnp.float32)],
    )(b0, b1p, b0w, b1w, rec2, ad2t, ones64, b2row)


def _iota16():
    return lax.iota(jnp.int32, 16)


def _lane_gather(v, idx):
    return lax.gather(
        v, idx[:, None],
        dimension_numbers=lax.GatherDimensionNumbers(
            offset_dims=(), collapsed_slice_dims=(0,), start_index_map=(0,)),
        slice_sizes=(1,),
        mode=lax.GatherScatterMode.PROMISE_IN_BOUNDS)


def _hsum_all(v):
    iot = _iota16()
    for off in (8, 4, 2, 1):
        v = v + _lane_gather(v, iot ^ off)
    return v


@functools.lru_cache(maxsize=None)
def _sc_edge_kernel(heads):
    @functools.partial(
        pl.kernel,
        out_type=[
            jax.ShapeDtypeStruct((N, R), jnp.float32),
            jax.ShapeDtypeStruct((N, R), jnp.float32),
            jax.ShapeDtypeStruct((N, 16), jnp.float32),
            jax.ShapeDtypeStruct((N, 16), jnp.float32),
        ],
        mesh=_mesh(),
        compiler_params=pltpu.CompilerParams(use_tc_tiling_on_sc=False),
        scratch_types=[
            pltpu.VMEM((QE, KE), jnp.int32),
            pltpu.VMEM((QE, KE), jnp.int32),
            pltpu.VMEM((KE, R), jnp.float32),
            pltpu.VMEM((KE, R), jnp.float32),
            pltpu.VMEM((KE, 16), jnp.float32),
            pltpu.VMEM((KE, 16), jnp.float32),
            pltpu.VMEM((KE, 16), jnp.float32),
            pltpu.VMEM((KE, 16), jnp.float32),
            pltpu.VMEM_SHARED((N, R), jnp.float32),
            pltpu.VMEM_SHARED((N, 16), jnp.float32),
            pltpu.SemaphoreType.DMA,
            pltpu.SemaphoreType.DMA,
            pltpu.SemaphoreType.DMA,
            pltpu.SemaphoreType.DMA,
        ],
    )
    def sc_edges(src2d_hbm, dst2d_hbm, rec_hbm, adt_hbm, zr_hbm, zw_hbm,
                 accf0_hbm, accf1_hbm, accw0_hbm, accw1_hbm,
                 sbig, dbig, recb0, recb1, adb0, adb1, wb0, wb1,
                 accf_sp, accw_sp, g0, g1, s0, s1):
        c = lax.axis_index("c")
        s = lax.axis_index("s")
        tid = s * NC + c

        recb = (recb0, recb1)
        adb = (adb0, adb1)
        wb = (wb0, wb1)
        gsem = (g0, g1)
        ssem = (s0, s1)

        pltpu.sync_copy(src2d_hbm.at[pl.ds(tid * QE, QE)], sbig)
        pltpu.sync_copy(dst2d_hbm.at[pl.ds(tid * QE, QE)], dbig)

        rpt = 624
        tail0 = rpt * NS
        r0 = s * rpt
        pltpu.sync_copy(zr_hbm.at[pl.ds(r0, rpt)], accf_sp.at[pl.ds(r0, rpt)])
        pltpu.sync_copy(zw_hbm.at[pl.ds(r0, rpt)], accw_sp.at[pl.ds(r0, rpt)])

        @pl.when(s == NS - 1)
        def _():
            pltpu.sync_copy(zr_hbm.at[pl.ds(tail0, N - tail0)],
                            accf_sp.at[pl.ds(tail0, N - tail0)])
            pltpu.sync_copy(zw_hbm.at[pl.ds(tail0, N - tail0)],
                            accw_sp.at[pl.ds(tail0, N - tail0)])

        plsc.subcore_barrier()

        iot = _iota16()
        pat8 = iot & 7

        def issue_gathers(q, b):
            pltpu.async_copy(rec_hbm.at[sbig.at[q]], recb[b], gsem[b])
            pltpu.async_copy(adt_hbm.at[dbig.at[q]], adb[b], gsem[b])

        def wait_gathers(b):
            pltpu.make_async_copy(rec_hbm.at[sbig.at[0]], recb[b],
                                  gsem[b]).wait()
            pltpu.make_async_copy(adt_hbm.at[dbig.at[0]], adb[b],
                                  gsem[b]).wait()

        def issue_scatters(q, b):
            pltpu.async_copy(recb[b], accf_sp.at[dbig.at[q]], ssem[b],
                             add=True)
            pltpu.async_copy(wb[b], accw_sp.at[dbig.at[q]], ssem[b], add=True)

        def wait_scatters(b):
            pltpu.make_async_copy(recb[b], accf_sp.at[dbig.at[0]],
                                  ssem[b]).wait()
            pltpu.make_async_copy(wb[b], accw_sp.at[dbig.at[0]],
                                  ssem[b]).wait()

        def compute(b):
            rb = recb[b]
            ab = adb[b]
            wbb = wb[b]

            def edge_body(k, carry):
                alpha = rb[k, pl.ds(64, 16)] + ab[k, :]
                w = jnp.exp(_leaky(alpha))
                wbb[k, :] = w
                wv = _lane_gather(w, pat8) if heads == 8 else w
                for j in range(4):
                    rb[k, pl.ds(16 * j, 16)] = rb[k, pl.ds(16 * j, 16)] * wv
                return carry

            lax.fori_loop(0, KE, edge_body, 0, unroll=4)

        def half(q, b):
            @pl.when(q >= 1)
            def _():
                wait_scatters(1 - b)

            @pl.when(q + 1 < QE)
            def _():
                issue_gathers(q + 1, 1 - b)

            wait_gathers(b)
            compute(b)
            issue_scatters(q, b)

        issue_gathers(0, 0)

        def pair_body(p, carry):
            half(2 * p, 0)
            half(2 * p + 1, 1)
            return carry

        lax.fori_loop(0, QE // 2, pair_body, 0)
        half(QE - 1, 0)
        wait_scatters(0)

        plsc.subcore_barrier()

        def dump(sp_ref, hbm_ref):
            pltpu.sync_copy(sp_ref.at[pl.ds(r0, rpt)],
                            hbm_ref.at[pl.ds(r0, rpt)])

            @pl.when(s == NS - 1)
            def _():
                pltpu.sync_copy(sp_ref.at[pl.ds(tail0, N - tail0)],
                                hbm_ref.at[pl.ds(tail0, N - tail0)])

        @pl.when(c == 0)
        def _():
            dump(accf_sp, accf0_hbm)
            dump(accw_sp, accw0_hbm)

        @pl.when(c == 1)
        def _():
            dump(accf_sp, accf1_hbm)
            dump(accw_sp, accw1_hbm)

    return sc_edges


@functools.lru_cache(maxsize=None)
def _sc_decode_kernel():
    @functools.partial(
        pl.kernel,
        out_type=jax.ShapeDtypeStruct((T,), jnp.float32),
        mesh=_mesh(),
        compiler_params=pltpu.CompilerParams(use_tc_tiling_on_sc=False),
        scratch_types=[
            pltpu.VMEM((QD, KD), jnp.int32),
            pltpu.VMEM((QD, KD), jnp.int32),
            pltpu.VMEM((KD, F2), jnp.float32),
            pltpu.VMEM((KD, F2), jnp.float32),
            pltpu.VMEM((KD, F2), jnp.float32),
            pltpu.VMEM((KD, F2), jnp.float32),
            pltpu.VMEM((KD,), jnp.float32),
            pltpu.VMEM((KD,), jnp.float32),
            pltpu.SemaphoreType.DMA,
            pltpu.SemaphoreType.DMA,
            pltpu.SemaphoreType.DMA,
            pltpu.SemaphoreType.DMA,
        ],
    )
    def sc_decode(z2_hbm, i0p_hbm, i1p_hbm, out_hbm,
                  ibig0, ibig1, za0, za1, zb0, zb1, ob0, ob1,
                  g0, g1, s0, s1):
        c = lax.axis_index("c")
        s = lax.axis_index("s")
        tid = s * NC + c

        za = (za0, za1)
        zb = (zb0, zb1)
        ob = (ob0, ob1)
        gsem = (g0, g1)
        ssem = (s0, s1)

        base_cnt = QD_TOT // NW
        rem = QD_TOT % NW
        cnt = base_cnt + jnp.where(tid < rem, 1, 0)
        start = base_cnt * tid + jnp.minimum(tid, rem)

        pltpu.sync_copy(i0p_hbm.at[pl.ds(start, QD)], ibig0)
        pltpu.sync_copy(i1p_hbm.at[pl.ds(start, QD)], ibig1)

        iot = _iota16()

        def issue_gathers(q, b):
            pltpu.async_copy(z2_hbm.at[ibig0.at[q]], za[b], gsem[b])
            pltpu.async_copy(z2_hbm.at[ibig1.at[q]], zb[b], gsem[b])

        def wait_gathers(b):
            pltpu.make_async_copy(z2_hbm.at[ibig0.at[0]], za[b],
                                  gsem[b]).wait()
            pltpu.make_async_copy(z2_hbm.at[ibig1.at[0]], zb[b],
                                  gsem[b]).wait()

        def compute(b):
            zab = za[b]
            zbb = zb[b]
            obb = ob[b]

            def group_body(g2, carry):
                def edge_body(e, gacc):
                    k = g2 * 16 + e
                    acc = zab[k, pl.ds(0, 16)] * zbb[k, pl.ds(0, 16)]
                    for j in range(1, 4):
                        acc = acc + (zab[k, pl.ds(16 * j, 16)]
                                     * zbb[k, pl.ds(16 * j, 16)])
                    return jnp.where(iot == e, _hsum_all(acc), gacc)

                gacc = lax.fori_loop(0, 16, edge_body,
                                     jnp.zeros((16,), jnp.float32), unroll=4)
                obb[pl.ds(g2 * 16, 16)] = gacc
                return carry

            lax.fori_loop(0, KD // 16, group_body, 0)

        def half(q, b):
            @pl.when(q + 1 < cnt)
            def _():
                issue_gathers(q + 1, 1 - b)

            wait_gathers(b)

            @pl.when(q >= 2)
            def _():
                pltpu.make_async_copy(
                    ob[b], out_hbm.at[pl.ds(0, KD)], ssem[b]).wait()

            compute(b)
            pltpu.async_copy(ob[b], out_hbm.at[pl.ds((start + q) * KD, KD)],
                             ssem[b])

        issue_gathers(0, 0)

        def pair_body(p, carry):
            @pl.when(2 * p < cnt)
            def _():
                half(2 * p, 0)

            @pl.when(2 * p + 1 < cnt)
            def _():
                half(2 * p + 1, 1)

            return carry

        lax.fori_loop(0, QD // 2, pair_body, 0)

        pltpu.make_async_copy(ob[0], out_hbm.at[pl.ds(0, KD)], ssem[0]).wait()
        pltpu.make_async_copy(ob[1], out_hbm.at[pl.ds(0, KD)], ssem[1]).wait()

    return sc_decode


def kernel(x, train_pos_edge_index, test_pos_edge_index, test_neg_edge_index,
           W1, a_src1, a_dst1, b1, W2, a_src2, a_dst2, b2):
    f32 = jnp.float32
    src2d = train_pos_edge_index[0].reshape(E // KE, KE)
    dst2d = train_pos_edge_index[1].reshape(E // KE, KE)
    ei = jnp.concatenate([test_pos_edge_index, test_neg_edge_index], axis=-1)
    npad = QD_TOT * KD // KD
    i0p = jnp.zeros((TPAD // KD, KD), ei.dtype).at[:npad].set(
        ei[0].reshape(npad, KD))
    i1p = jnp.zeros((TPAD // KD, KD), ei.dtype).at[:npad].set(
        ei[1].reshape(npad, KD))

    cols = np.arange(F1)
    perm = (cols % 8) * 8 + cols // 8
    pp = jnp.zeros((F1, F1), f32).at[perm, cols].set(1.0)
    pm = jnp.zeros((F1, F1), f32).at[cols, perm].set(1.0)
    hh = jnp.repeat(jnp.arange(H1), C1)
    asrc_bd = jnp.zeros((F1, 8), f32).at[jnp.arange(F1), hh].set(
        a_src1.reshape(F1))
    adst_bd = jnp.zeros((F1, 8), f32).at[jnp.arange(F1), hh].set(
        a_dst1.reshape(F1))
    p1 = jnp.concatenate([pp, asrc_bd, jnp.zeros((F1, 8), f32)], axis=1)
    ad1m = jnp.concatenate([adst_bd, jnp.zeros((F1, 8), f32)], axis=1)
    b8il = jnp.zeros((8, F1), f32).at[cols % 8, cols].set(1.0)

    p2 = jnp.concatenate(
        [jnp.eye(F2, dtype=f32),
         jnp.outer(a_src2.reshape(F2), jnp.ones((16,), f32))], axis=1)
    ad2m = jnp.outer(a_dst2.reshape(F2), jnp.ones((16,), f32))
    ones64 = jnp.ones((1, F2), f32)

    zr = jnp.zeros((N, R), f32)
    zw = jnp.zeros((N, 16), f32)

    rec1, ad1t = _tc_stage_a(x, W1.T, p1, ad1m)
    a0, a1, a0w, a1w = _sc_edge_kernel(8)(src2d, dst2d, rec1, ad1t, zr, zw)
    rec2, ad2t = _tc_stage_b(a0, a1, a0w, a1w, rec1, ad1t, W2.T, p2, ad2m,
                             b8il, pm, b1.reshape(1, F1))
    b0, b1p, b0w, b1w = _sc_edge_kernel(1)(src2d, dst2d, rec2, ad2t, zr, zw)
    (z2,) = _tc_stage_c(b0, b1p, b0w, b1w, rec2, ad2t, ones64,
                        b2.reshape(1, F2))
    logits = _sc_decode_kernel()(z2, i0p, i1p)
    return logits


# speedup vs baseline: 2.1265x; 1.6344x over previous
import functools

import jax
import jax.numpy as jnp
import numpy as np
from jax import lax
from jax.experimental import pallas as pl
from jax.experimental.pallas import tpu as pltpu
from jax.experimental.pallas import tpu_sc as plsc

N = 10000
D = 128
E = 320000
T = 80000
H1, C1, F1 = 8, 8, 64
F2 = 64

NC, NS = 2, 16
NW = NC * NS
R = 80

EPT = E // NW
KE = 80
QE = EPT // KE

KD = 128
QD_TOT = T // KD
QD = 20
TPAD = NW * QD * KD


def _mesh():
    return plsc.VectorSubcoreMesh(
        core_axis_name="c", subcore_axis_name="s",
        num_cores=NC, num_subcores=NS)


def _leaky(x):
    return jnp.maximum(x, 0.2 * x)


def _tca_body(x_ref, w1t_ref, p1_ref, ad1m_ref, rec1_ref, ad1t_ref):
    xp = jnp.dot(x_ref[...], w1t_ref[...], preferred_element_type=jnp.float32)
    rec1_ref[...] = jnp.dot(xp, p1_ref[...], preferred_element_type=jnp.float32)
    ad1t_ref[...] = jnp.dot(xp, ad1m_ref[...],
                            preferred_element_type=jnp.float32)


def _tc_stage_a(x, w1t, p1, ad1m):
    blk = 1000
    return pl.pallas_call(
        _tca_body,
        grid=(N // blk,),
        in_specs=[
            pl.BlockSpec((blk, D), lambda i: (i, 0)),
            pl.BlockSpec((D, F1), lambda i: (0, 0)),
            pl.BlockSpec((F1, R), lambda i: (0, 0)),
            pl.BlockSpec((F1, 16), lambda i: (0, 0)),
        ],
        out_specs=[
            pl.BlockSpec((blk, R), lambda i: (i, 0)),
            pl.BlockSpec((blk, 16), lambda i: (i, 0)),
        ],
        out_shape=[
            jax.ShapeDtypeStruct((N, R), jnp.float32),
            jax.ShapeDtypeStruct((N, 16), jnp.float32),
        ],
    )(x, w1t, p1, ad1m)


def _tcb_body(a0_ref, a1_ref, a0w_ref, a1w_ref, rec1_ref, ad1t_ref, w2t_ref,
              p2_ref, ad2m_ref, b8il_ref, pm_ref, b1_ref, rec2_ref, ad2t_ref):
    xp_il = rec1_ref[:, 0:F1]
    asrc = rec1_ref[:, 64:72]
    adst = ad1t_ref[:, 0:8]
    wself = jnp.exp(_leaky(asrc + adst))
    b8il = b8il_ref[...]
    numer = (a0_ref[:, 0:F1] + a1_ref[:, 0:F1]
             + xp_il * jnp.dot(wself, b8il, preferred_element_type=jnp.float32))
    den8 = a0w_ref[:, 0:8] + a1w_ref[:, 0:8] + wself
    z_il = numer / jnp.dot(den8, b8il, preferred_element_type=jnp.float32)
    z = jnp.dot(z_il, pm_ref[...], preferred_element_type=jnp.float32) \
        + b1_ref[...]
    z = jnp.where(z > 0, z, jnp.exp(z) - 1.0)
    xp2 = jnp.dot(z, w2t_ref[...], preferred_element_type=jnp.float32)
    rec2_ref[...] = jnp.dot(xp2, p2_ref[...],
                            preferred_element_type=jnp.float32)
    ad2t_ref[...] = jnp.dot(xp2, ad2m_ref[...],
                            preferred_element_type=jnp.float32)


def _tc_stage_b(a0, a1, a0w, a1w, rec1, ad1t, w2t, p2, ad2m, b8il, pm, b1row):
    blk = 1000
    return pl.pallas_call(
        _tcb_body,
        grid=(N // blk,),
        in_specs=[
            pl.BlockSpec((blk, R), lambda i: (i, 0)),
            pl.BlockSpec((blk, R), lambda i: (i, 0)),
            pl.BlockSpec((blk, 16), lambda i: (i, 0)),
            pl.BlockSpec((blk, 16), lambda i: (i, 0)),
            pl.BlockSpec((blk, R), lambda i: (i, 0)),
            pl.BlockSpec((blk, 16), lambda i: (i, 0)),
            pl.BlockSpec((F1, F2), lambda i: (0, 0)),
            pl.BlockSpec((F2, R), lambda i: (0, 0)),
            pl.BlockSpec((F2, 16), lambda i: (0, 0)),
            pl.BlockSpec((8, F1), lambda i: (0, 0)),
            pl.BlockSpec((F1, F1), lambda i: (0, 0)),
            pl.BlockSpec((1, F1), lambda i: (0, 0)),
        ],
        out_specs=[
            pl.BlockSpec((blk, R), lambda i: (i, 0)),
            pl.BlockSpec((blk, 16), lambda i: (i, 0)),
        ],
        out_shape=[
            jax.ShapeDtypeStruct((N, R), jnp.float32),
            jax.ShapeDtypeStruct((N, 16), jnp.float32),
        ],
    )(a0, a1, a0w, a1w, rec1, ad1t, w2t, p2, ad2m, b8il, pm, b1row)


def _tcc_body(b0_ref, b1p_ref, b0w_ref, b1w_ref, rec2_ref, ad2t_ref,
              ones64_ref, b2_ref, z2_ref):
    xp2 = rec2_ref[:, 0:F2]
    as2 = rec2_ref[:, 64:65]
    ad2 = ad2t_ref[:, 0:1]
    w2 = jnp.exp(_leaky(as2 + ad2))
    ones64 = ones64_ref[...]
    numer = (b0_ref[:, 0:F2] + b1p_ref[:, 0:F2]
             + xp2 * jnp.dot(w2, ones64, preferred_element_type=jnp.float32))
    denom = jnp.dot(b0w_ref[:, 0:1] + b1w_ref[:, 0:1] + w2, ones64,
                    preferred_element_type=jnp.float32)
    z2_ref[...] = numer / denom + b2_ref[...]


def _tc_stage_c(b0, b1p, b0w, b1w, rec2, ad2t, ones64, b2row):
    blk = 1000
    return pl.pallas_call(
        _tcc_body,
        grid=(N // blk,),
        in_specs=[
            pl.BlockSpec((blk, R), lambda i: (i, 0)),
            pl.BlockSpec((blk, R), lambda i: (i, 0)),
            pl.BlockSpec((blk, 16), lambda i: (i, 0)),
            pl.BlockSpec((blk, 16), lambda i: (i, 0)),
            pl.BlockSpec((blk, R), lambda i: (i, 0)),
            pl.BlockSpec((blk, 16), lambda i: (i, 0)),
            pl.BlockSpec((1, F2), lambda i: (0, 0)),
            pl.BlockSpec((1, F2), lambda i: (0, 0)),
        ],
        out_specs=[pl.BlockSpec((blk, F2), lambda i: (i, 0))],
        out_shape=[jax.ShapeDtypeStruct((N, F2), jnp.float32)],
    )(b0, b1p, b0w, b1w, rec2, ad2t, ones64, b2row)


def _iota16():
    return lax.iota(jnp.int32, 16)


def _lane_gather(v, idx):
    return lax.gather(
        v, idx[:, None],
        dimension_numbers=lax.GatherDimensionNumbers(
            offset_dims=(), collapsed_slice_dims=(0,), start_index_map=(0,)),
        slice_sizes=(1,),
        mode=lax.GatherScatterMode.PROMISE_IN_BOUNDS)


def _hsum_all(v):
    iot = _iota16()
    for off in (8, 4, 2, 1):
        v = v + _lane_gather(v, iot ^ off)
    return v


@functools.lru_cache(maxsize=None)
def _sc_edge_kernel(heads):
    @functools.partial(
        pl.kernel,
        out_type=[
            jax.ShapeDtypeStruct((N, R), jnp.float32),
            jax.ShapeDtypeStruct((N, R), jnp.float32),
            jax.ShapeDtypeStruct((N, 16), jnp.float32),
            jax.ShapeDtypeStruct((N, 16), jnp.float32),
        ],
        mesh=_mesh(),
        compiler_params=pltpu.CompilerParams(use_tc_tiling_on_sc=False),
        scratch_types=[
            pltpu.VMEM((QE, KE), jnp.int32),
            pltpu.VMEM((QE, KE), jnp.int32),
            pltpu.VMEM((KE, R), jnp.float32),
            pltpu.VMEM((KE, R), jnp.float32),
            pltpu.VMEM((KE, 16), jnp.float32),
            pltpu.VMEM((KE, 16), jnp.float32),
            pltpu.VMEM((KE, 16), jnp.float32),
            pltpu.VMEM((KE, 16), jnp.float32),
            pltpu.VMEM_SHARED((N, R), jnp.float32),
            pltpu.VMEM_SHARED((N, 16), jnp.float32),
            pltpu.SemaphoreType.DMA,
            pltpu.SemaphoreType.DMA,
            pltpu.SemaphoreType.DMA,
            pltpu.SemaphoreType.DMA,
        ],
    )
    def sc_edges(src2d_hbm, dst2d_hbm, rec_hbm, adt_hbm, zr_hbm, zw_hbm,
                 accf0_hbm, accf1_hbm, accw0_hbm, accw1_hbm,
                 sbig, dbig, recb0, recb1, adb0, adb1, wb0, wb1,
                 accf_sp, accw_sp, g0, g1, s0, s1):
        c = lax.axis_index("c")
        s = lax.axis_index("s")
        tid = s * NC + c

        recb = (recb0, recb1)
        adb = (adb0, adb1)
        wb = (wb0, wb1)
        gsem = (g0, g1)
        ssem = (s0, s1)

        pltpu.sync_copy(src2d_hbm.at[pl.ds(tid * QE, QE)], sbig)
        pltpu.sync_copy(dst2d_hbm.at[pl.ds(tid * QE, QE)], dbig)

        rpt = 624
        tail0 = rpt * NS
        r0 = s * rpt
        pltpu.sync_copy(zr_hbm.at[pl.ds(r0, rpt)], accf_sp.at[pl.ds(r0, rpt)])
        pltpu.sync_copy(zw_hbm.at[pl.ds(r0, rpt)], accw_sp.at[pl.ds(r0, rpt)])

        @pl.when(s == NS - 1)
        def _():
            pltpu.sync_copy(zr_hbm.at[pl.ds(tail0, N - tail0)],
                            accf_sp.at[pl.ds(tail0, N - tail0)])
            pltpu.sync_copy(zw_hbm.at[pl.ds(tail0, N - tail0)],
                            accw_sp.at[pl.ds(tail0, N - tail0)])

        plsc.subcore_barrier()

        iot = _iota16()
        pat8 = iot & 7

        def issue_gathers(q, b):
            pltpu.async_copy(rec_hbm.at[sbig.at[q]], recb[b], gsem[b])
            pltpu.async_copy(adt_hbm.at[dbig.at[q]], adb[b], gsem[b])

        def wait_gathers(b):
            pltpu.make_async_copy(rec_hbm.at[sbig.at[0]], recb[b],
                                  gsem[b]).wait()
            pltpu.make_async_copy(adt_hbm.at[dbig.at[0]], adb[b],
                                  gsem[b]).wait()

        def issue_scatters(q, b):
            pltpu.async_copy(recb[b], accf_sp.at[dbig.at[q]], ssem[b],
                             add=True)
            pltpu.async_copy(wb[b], accw_sp.at[dbig.at[q]], ssem[b], add=True)

        def wait_scatters(b):
            pltpu.make_async_copy(recb[b], accf_sp.at[dbig.at[0]],
                                  ssem[b]).wait()
            pltpu.make_async_copy(wb[b], accw_sp.at[dbig.at[0]],
                                  ssem[b]).wait()

        def compute(b):
            rb = recb[b]
            ab = adb[b]
            wbb = wb[b]

            @plsc.parallel_loop(0, KE, unroll=4)
            def edge_body(k):
                alpha = rb[k, pl.ds(64, 16)] + ab[k, :]
                w = jnp.exp(_leaky(alpha))
                wbb[k, :] = w
                wv = _lane_gather(w, pat8) if heads == 8 else w
                for j in range(4):
                    rb[k, pl.ds(16 * j, 16)] = rb[k, pl.ds(16 * j, 16)] * wv

        def half(q, b):
            @pl.when(q >= 1)
            def _():
                wait_scatters(1 - b)

            @pl.when(q + 1 < QE)
            def _():
                issue_gathers(q + 1, 1 - b)

            wait_gathers(b)
            compute(b)
            issue_scatters(q, b)

        issue_gathers(0, 0)

        def pair_body(p, carry):
            half(2 * p, 0)
            half(2 * p + 1, 1)
            return carry

        lax.fori_loop(0, QE // 2, pair_body, 0)
        half(QE - 1, 0)
        wait_scatters(0)

        plsc.subcore_barrier()

        def dump(sp_ref, hbm_ref):
            pltpu.sync_copy(sp_ref.at[pl.ds(r0, rpt)],
                            hbm_ref.at[pl.ds(r0, rpt)])

            @pl.when(s == NS - 1)
            def _():
                pltpu.sync_copy(sp_ref.at[pl.ds(tail0, N - tail0)],
                                hbm_ref.at[pl.ds(tail0, N - tail0)])

        @pl.when(c == 0)
        def _():
            dump(accf_sp, accf0_hbm)
            dump(accw_sp, accw0_hbm)

        @pl.when(c == 1)
        def _():
            dump(accf_sp, accf1_hbm)
            dump(accw_sp, accw1_hbm)

    return sc_edges


@functools.lru_cache(maxsize=None)
def _sc_decode_kernel():
    @functools.partial(
        pl.kernel,
        out_type=jax.ShapeDtypeStruct((T,), jnp.float32),
        mesh=_mesh(),
        compiler_params=pltpu.CompilerParams(use_tc_tiling_on_sc=False),
        scratch_types=[
            pltpu.VMEM((QD, KD), jnp.int32),
            pltpu.VMEM((QD, KD), jnp.int32),
            pltpu.VMEM((KD, F2), jnp.float32),
            pltpu.VMEM((KD, F2), jnp.float32),
            pltpu.VMEM((KD, F2), jnp.float32),
            pltpu.VMEM((KD, F2), jnp.float32),
            pltpu.VMEM((KD,), jnp.float32),
            pltpu.VMEM((KD,), jnp.float32),
            pltpu.SemaphoreType.DMA,
            pltpu.SemaphoreType.DMA,
            pltpu.SemaphoreType.DMA,
            pltpu.SemaphoreType.DMA,
        ],
    )
    def sc_decode(z2_hbm, i0p_hbm, i1p_hbm, out_hbm,
                  ibig0, ibig1, za0, za1, zb0, zb1, ob0, ob1,
                  g0, g1, s0, s1):
        c = lax.axis_index("c")
        s = lax.axis_index("s")
        tid = s * NC + c

        za = (za0, za1)
        zb = (zb0, zb1)
        ob = (ob0, ob1)
        gsem = (g0, g1)
        ssem = (s0, s1)

        base_cnt = QD_TOT // NW
        rem = QD_TOT % NW
        cnt = base_cnt + jnp.where(tid < rem, 1, 0)
        start = base_cnt * tid + jnp.minimum(tid, rem)

        pltpu.sync_copy(i0p_hbm.at[pl.ds(start, QD)], ibig0)
        pltpu.sync_copy(i1p_hbm.at[pl.ds(start, QD)], ibig1)

        iot = _iota16()

        def issue_gathers(q, b):
            pltpu.async_copy(z2_hbm.at[ibig0.at[q]], za[b], gsem[b])
            pltpu.async_copy(z2_hbm.at[ibig1.at[q]], zb[b], gsem[b])

        def wait_gathers(b):
            pltpu.make_async_copy(z2_hbm.at[ibig0.at[0]], za[b],
                                  gsem[b]).wait()
            pltpu.make_async_copy(z2_hbm.at[ibig1.at[0]], zb[b],
                                  gsem[b]).wait()

        def compute(b):
            zab = za[b]
            zbb = zb[b]
            obb = ob[b]

            def group_body(g2, carry):
                def edge_body(e, gacc):
                    k = g2 * 16 + e
                    acc = zab[k, pl.ds(0, 16)] * zbb[k, pl.ds(0, 16)]
                    for j in range(1, 4):
                        acc = acc + (zab[k, pl.ds(16 * j, 16)]
                                     * zbb[k, pl.ds(16 * j, 16)])
                    return jnp.where(iot == e, _hsum_all(acc), gacc)

                gacc = lax.fori_loop(0, 16, edge_body,
                                     jnp.zeros((16,), jnp.float32), unroll=4)
                obb[pl.ds(g2 * 16, 16)] = gacc
                return carry

            lax.fori_loop(0, KD // 16, group_body, 0)

        def half(q, b):
            @pl.when(q + 1 < cnt)
            def _():
                issue_gathers(q + 1, 1 - b)

            wait_gathers(b)

            @pl.when(q >= 2)
            def _():
                pltpu.make_async_copy(
                    ob[b], out_hbm.at[pl.ds(0, KD)], ssem[b]).wait()

            compute(b)
            pltpu.async_copy(ob[b], out_hbm.at[pl.ds((start + q) * KD, KD)],
                             ssem[b])

        issue_gathers(0, 0)

        def pair_body(p, carry):
            @pl.when(2 * p < cnt)
            def _():
                half(2 * p, 0)

            @pl.when(2 * p + 1 < cnt)
            def _():
                half(2 * p + 1, 1)

            return carry

        lax.fori_loop(0, QD // 2, pair_body, 0)

        pltpu.make_async_copy(ob[0], out_hbm.at[pl.ds(0, KD)], ssem[0]).wait()
        pltpu.make_async_copy(ob[1], out_hbm.at[pl.ds(0, KD)], ssem[1]).wait()

    return sc_decode


def kernel(x, train_pos_edge_index, test_pos_edge_index, test_neg_edge_index,
           W1, a_src1, a_dst1, b1, W2, a_src2, a_dst2, b2):
    f32 = jnp.float32
    src2d = train_pos_edge_index[0].reshape(E // KE, KE)
    dst2d = train_pos_edge_index[1].reshape(E // KE, KE)
    ei = jnp.concatenate([test_pos_edge_index, test_neg_edge_index], axis=-1)
    npad = QD_TOT * KD // KD
    i0p = jnp.zeros((TPAD // KD, KD), ei.dtype).at[:npad].set(
        ei[0].reshape(npad, KD))
    i1p = jnp.zeros((TPAD // KD, KD), ei.dtype).at[:npad].set(
        ei[1].reshape(npad, KD))

    cols = np.arange(F1)
    perm = (cols % 8) * 8 + cols // 8
    pp = jnp.zeros((F1, F1), f32).at[perm, cols].set(1.0)
    pm = jnp.zeros((F1, F1), f32).at[cols, perm].set(1.0)
    hh = jnp.repeat(jnp.arange(H1), C1)
    asrc_bd = jnp.zeros((F1, 8), f32).at[jnp.arange(F1), hh].set(
        a_src1.reshape(F1))
    adst_bd = jnp.zeros((F1, 8), f32).at[jnp.arange(F1), hh].set(
        a_dst1.reshape(F1))
    p1 = jnp.concatenate([pp, asrc_bd, jnp.zeros((F1, 8), f32)], axis=1)
    ad1m = jnp.concatenate([adst_bd, jnp.zeros((F1, 8), f32)], axis=1)
    b8il = jnp.zeros((8, F1), f32).at[cols % 8, cols].set(1.0)

    p2 = jnp.concatenate(
        [jnp.eye(F2, dtype=f32),
         jnp.outer(a_src2.reshape(F2), jnp.ones((16,), f32))], axis=1)
    ad2m = jnp.outer(a_dst2.reshape(F2), jnp.ones((16,), f32))
    ones64 = jnp.ones((1, F2), f32)

    zr = jnp.zeros((N, R), f32)
    zw = jnp.zeros((N, 16), f32)

    rec1, ad1t = _tc_stage_a(x, W1.T, p1, ad1m)
    a0, a1, a0w, a1w = _sc_edge_kernel(8)(src2d, dst2d, rec1, ad1t, zr, zw)
    rec2, ad2t = _tc_stage_b(a0, a1, a0w, a1w, rec1, ad1t, W2.T, p2, ad2m,
                             b8il, pm, b1.reshape(1, F1))
    b0, b1p, b0w, b1w = _sc_edge_kernel(1)(src2d, dst2d, rec2, ad2t, zr, zw)
    (z2,) = _tc_stage_c(b0, b1p, b0w, b1w, rec2, ad2t, ones64,
                        b2.reshape(1, F2))
    logits = _sc_decode_kernel()(z2, i0p, i1p)
    return logits


# speedup vs baseline: 2.2752x; 1.0699x over previous
import functools

import jax
import jax.numpy as jnp
import numpy as np
from jax import lax
from jax.experimental import pallas as pl
from jax.experimental.pallas import tpu as pltpu
from jax.experimental.pallas import tpu_sc as plsc

N = 10000
D = 128
E = 320000
T = 80000
H1, C1, F1 = 8, 8, 64
F2 = 64

NC, NS = 2, 16
NW = NC * NS
R = 80

KE = 128
QE_TOT = E // KE
QE = QE_TOT // NW + 1
QE_PAD = 2504

KD = 128
QD_TOT = T // KD
QD = 20
TPAD = NW * QD * KD


def _mesh():
    return plsc.VectorSubcoreMesh(
        core_axis_name="c", subcore_axis_name="s",
        num_cores=NC, num_subcores=NS)


def _leaky(x):
    return jnp.maximum(x, 0.2 * x)


def _tca_body(x_ref, w1t_ref, p1_ref, ad1m_ref, rec1_ref, ad1t_ref):
    xp = jnp.dot(x_ref[...], w1t_ref[...], preferred_element_type=jnp.float32)
    rec1_ref[...] = jnp.dot(xp, p1_ref[...], preferred_element_type=jnp.float32)
    ad1t_ref[...] = jnp.dot(xp, ad1m_ref[...],
                            preferred_element_type=jnp.float32)


def _tc_stage_a(x, w1t, p1, ad1m):
    blk = 1000
    return pl.pallas_call(
        _tca_body,
        grid=(N // blk,),
        in_specs=[
            pl.BlockSpec((blk, D), lambda i: (i, 0)),
            pl.BlockSpec((D, F1), lambda i: (0, 0)),
            pl.BlockSpec((F1, R), lambda i: (0, 0)),
            pl.BlockSpec((F1, 16), lambda i: (0, 0)),
        ],
        out_specs=[
            pl.BlockSpec((blk, R), lambda i: (i, 0)),
            pl.BlockSpec((blk, 16), lambda i: (i, 0)),
        ],
        out_shape=[
            jax.ShapeDtypeStruct((N, R), jnp.float32),
            jax.ShapeDtypeStruct((N, 16), jnp.float32),
        ],
    )(x, w1t, p1, ad1m)


def _tcb_body(a0_ref, a1_ref, a0w_ref, a1w_ref, rec1_ref, ad1t_ref, w2t_ref,
              p2_ref, ad2m_ref, b8il_ref, pm_ref, b1_ref, rec2_ref, ad2t_ref):
    xp_il = rec1_ref[:, 0:F1]
    asrc = rec1_ref[:, 64:72]
    adst = ad1t_ref[:, 0:8]
    wself = jnp.exp(_leaky(asrc + adst))
    b8il = b8il_ref[...]
    numer = (a0_ref[:, 0:F1] + a1_ref[:, 0:F1]
             + xp_il * jnp.dot(wself, b8il, preferred_element_type=jnp.float32))
    den8 = a0w_ref[:, 0:8] + a1w_ref[:, 0:8] + wself
    z_il = numer / jnp.dot(den8, b8il, preferred_element_type=jnp.float32)
    z = jnp.dot(z_il, pm_ref[...], preferred_element_type=jnp.float32) \
        + b1_ref[...]
    z = jnp.where(z > 0, z, jnp.exp(z) - 1.0)
    xp2 = jnp.dot(z, w2t_ref[...], preferred_element_type=jnp.float32)
    rec2_ref[...] = jnp.dot(xp2, p2_ref[...],
                            preferred_element_type=jnp.float32)
    ad2t_ref[...] = jnp.dot(xp2, ad2m_ref[...],
                            preferred_element_type=jnp.float32)


def _tc_stage_b(a0, a1, a0w, a1w, rec1, ad1t, w2t, p2, ad2m, b8il, pm, b1row):
    blk = 1000
    return pl.pallas_call(
        _tcb_body,
        grid=(N // blk,),
        in_specs=[
            pl.BlockSpec((blk, R), lambda i: (i, 0)),
            pl.BlockSpec((blk, R), lambda i: (i, 0)),
            pl.BlockSpec((blk, 16), lambda i: (i, 0)),
            pl.BlockSpec((blk, 16), lambda i: (i, 0)),
            pl.BlockSpec((blk, R), lambda i: (i, 0)),
            pl.BlockSpec((blk, 16), lambda i: (i, 0)),
            pl.BlockSpec((F1, F2), lambda i: (0, 0)),
            pl.BlockSpec((F2, R), lambda i: (0, 0)),
            pl.BlockSpec((F2, 16), lambda i: (0, 0)),
            pl.BlockSpec((8, F1), lambda i: (0, 0)),
            pl.BlockSpec((F1, F1), lambda i: (0, 0)),
            pl.BlockSpec((1, F1), lambda i: (0, 0)),
        ],
        out_specs=[
            pl.BlockSpec((blk, R), lambda i: (i, 0)),
            pl.BlockSpec((blk, 16), lambda i: (i, 0)),
        ],
        out_shape=[
            jax.ShapeDtypeStruct((N, R), jnp.float32),
            jax.ShapeDtypeStruct((N, 16), jnp.float32),
        ],
    )(a0, a1, a0w, a1w, rec1, ad1t, w2t, p2, ad2m, b8il, pm, b1row)


def _tcc_body(b0_ref, b1p_ref, b0w_ref, b1w_ref, rec2_ref, ad2t_ref,
              ones64_ref, b2_ref, z2_ref):
    xp2 = rec2_ref[:, 0:F2]
    as2 = rec2_ref[:, 64:65]
    ad2 = ad2t_ref[:, 0:1]
    w2 = jnp.exp(_leaky(as2 + ad2))
    ones64 = ones64_ref[...]
    numer = (b0_ref[:, 0:F2] + b1p_ref[:, 0:F2]
             + xp2 * jnp.dot(w2, ones64, preferred_element_type=jnp.float32))
    denom = jnp.dot(b0w_ref[:, 0:1] + b1w_ref[:, 0:1] + w2, ones64,
                    preferred_element_type=jnp.float32)
    z2_ref[...] = numer / denom + b2_ref[...]


def _tc_stage_c(b0, b1p, b0w, b1w, rec2, ad2t, ones64, b2row):
    blk = 1000
    return pl.pallas_call(
        _tcc_body,
        grid=(N // blk,),
        in_specs=[
            pl.BlockSpec((blk, R), lambda i: (i, 0)),
            pl.BlockSpec((blk, R), lambda i: (i, 0)),
            pl.BlockSpec((blk, 16), lambda i: (i, 0)),
            pl.BlockSpec((blk, 16), lambda i: (i, 0)),
            pl.BlockSpec((blk, R), lambda i: (i, 0)),
            pl.BlockSpec((blk, 16), lambda i: (i, 0)),
            pl.BlockSpec((1, F2), lambda i: (0, 0)),
            pl.BlockSpec((1, F2), lambda i: (0, 0)),
        ],
        out_specs=[pl.BlockSpec((blk, F2), lambda i: (i, 0))],
        out_shape=[jax.ShapeDtypeStruct((N, F2), jnp.float32)],
    )(b0, b1p, b0w, b1w, rec2, ad2t, ones64, b2row)


def _iota16():
    return lax.iota(jnp.int32, 16)


def _lane_gather(v, idx):
    return lax.gather(
        v, idx[:, None],
        dimension_numbers=lax.GatherDimensionNumbers(
            offset_dims=(), collapsed_slice_dims=(0,), start_index_map=(0,)),
        slice_sizes=(1,),
        mode=lax.GatherScatterMode.PROMISE_IN_BOUNDS)


def _hsum_all(v):
    iot = _iota16()
    for off in (8, 4, 2, 1):
        v = v + _lane_gather(v, iot ^ off)
    return v


@functools.lru_cache(maxsize=None)
def _sc_edge_kernel(heads):
    @functools.partial(
        pl.kernel,
        out_type=[
            jax.ShapeDtypeStruct((N, R), jnp.float32),
            jax.ShapeDtypeStruct((N, R), jnp.float32),
            jax.ShapeDtypeStruct((N, 16), jnp.float32),
            jax.ShapeDtypeStruct((N, 16), jnp.float32),
        ],
        mesh=_mesh(),
        compiler_params=pltpu.CompilerParams(use_tc_tiling_on_sc=False),
        scratch_types=[
            pltpu.VMEM((QE, KE), jnp.int32),
            pltpu.VMEM((QE, KE), jnp.int32),

            pltpu.VMEM((KE, R), jnp.float32),
            pltpu.VMEM((KE, R), jnp.float32),
            pltpu.VMEM((KE, 16), jnp.float32),
            pltpu.VMEM((KE, 16), jnp.float32),
            pltpu.VMEM((KE, 16), jnp.float32),
            pltpu.VMEM((KE, 16), jnp.float32),
            pltpu.VMEM_SHARED((N, R), jnp.float32),
            pltpu.VMEM_SHARED((N, 16), jnp.float32),
            pltpu.SemaphoreType.DMA,
            pltpu.SemaphoreType.DMA,
            pltpu.SemaphoreType.DMA,
            pltpu.SemaphoreType.DMA,
        ],
    )
    def sc_edges(src2d_hbm, dst2d_hbm, rec_hbm, adt_hbm, zr_hbm, zw_hbm,
                 accf0_hbm, accf1_hbm, accw0_hbm, accw1_hbm,
                 sbig, dbig, recb0, recb1, adb0, adb1, wb0, wb1,
                 accf_sp, accw_sp, g0, g1, s0, s1):
        c = lax.axis_index("c")
        s = lax.axis_index("s")
        tid = s * NC + c

        recb = (recb0, recb1)
        adb = (adb0, adb1)
        wb = (wb0, wb1)
        gsem = (g0, g1)
        ssem = (s0, s1)

        base_cnt = QE_TOT // NW
        rem = QE_TOT % NW
        cnt = base_cnt + jnp.where(tid < rem, 1, 0)
        start = base_cnt * tid + jnp.minimum(tid, rem)

        pltpu.sync_copy(src2d_hbm.at[pl.ds(start, QE)], sbig)
        pltpu.sync_copy(dst2d_hbm.at[pl.ds(start, QE)], dbig)

        rpt = 624
        tail0 = rpt * NS
        r0 = s * rpt
        pltpu.sync_copy(zr_hbm.at[pl.ds(r0, rpt)], accf_sp.at[pl.ds(r0, rpt)])
        pltpu.sync_copy(zw_hbm.at[pl.ds(r0, rpt)], accw_sp.at[pl.ds(r0, rpt)])

        @pl.when(s == NS - 1)
        def _():
            pltpu.sync_copy(zr_hbm.at[pl.ds(tail0, N - tail0)],
                            accf_sp.at[pl.ds(tail0, N - tail0)])
            pltpu.sync_copy(zw_hbm.at[pl.ds(tail0, N - tail0)],
                            accw_sp.at[pl.ds(tail0, N - tail0)])

        plsc.subcore_barrier()

        iot = _iota16()
        pat8 = iot & 7

        def issue_gathers(q, b):
            pltpu.async_copy(rec_hbm.at[sbig.at[q]], recb[b], gsem[b])
            pltpu.async_copy(adt_hbm.at[dbig.at[q]], adb[b], gsem[b])

        def wait_gathers(b):
            pltpu.make_async_copy(rec_hbm.at[sbig.at[0]], recb[b],
                                  gsem[b]).wait()
            pltpu.make_async_copy(adt_hbm.at[dbig.at[0]], adb[b],
                                  gsem[b]).wait()

        def issue_scatters(q, b):
            pltpu.async_copy(recb[b], accf_sp.at[dbig.at[q]], ssem[b],
                             add=True)
            pltpu.async_copy(wb[b], accw_sp.at[dbig.at[q]], ssem[b], add=True)

        def wait_scatters(b):
            pltpu.make_async_copy(recb[b], accf_sp.at[dbig.at[0]],
                                  ssem[b]).wait()
            pltpu.make_async_copy(wb[b], accw_sp.at[dbig.at[0]],
                                  ssem[b]).wait()

        def compute(b):
            rb = recb[b]
            ab = adb[b]
            wbb = wb[b]

            @plsc.parallel_loop(0, KE, unroll=4)
            def edge_body(k):
                alpha = rb[k, pl.ds(64, 16)] + ab[k, :]
                w = jnp.exp(_leaky(alpha))
                wbb[k, :] = w
                wv = _lane_gather(w, pat8) if heads == 8 else w
                for j in range(4):
                    rb[k, pl.ds(16 * j, 16)] = rb[k, pl.ds(16 * j, 16)] * wv

        def half(q, b):
            @pl.when(q >= 1)
            def _():
                wait_scatters(1 - b)

            @pl.when(q + 1 < cnt)
            def _():
                issue_gathers(q + 1, 1 - b)

            wait_gathers(b)
            compute(b)
            issue_scatters(q, b)

        issue_gathers(0, 0)

        def pair_body(p, carry):
            @pl.when(2 * p < cnt)
            def _():
                half(2 * p, 0)

            @pl.when(2 * p + 1 < cnt)
            def _():
                half(2 * p + 1, 1)

            return carry

        lax.fori_loop(0, QE // 2 + 1, pair_body, 0)
        @pl.when(cnt % 2 == 1)
        def _():
            wait_scatters(0)

        @pl.when(cnt % 2 == 0)
        def _():
            wait_scatters(1)

        plsc.subcore_barrier()

        def dump(sp_ref, hbm_ref):
            pltpu.sync_copy(sp_ref.at[pl.ds(r0, rpt)],
                            hbm_ref.at[pl.ds(r0, rpt)])

            @pl.when(s == NS - 1)
            def _():
                pltpu.sync_copy(sp_ref.at[pl.ds(tail0, N - tail0)],
                                hbm_ref.at[pl.ds(tail0, N - tail0)])

        @pl.when(c == 0)
        def _():
            dump(accf_sp, accf0_hbm)
            dump(accw_sp, accw0_hbm)

        @pl.when(c == 1)
        def _():
            dump(accf_sp, accf1_hbm)
            dump(accw_sp, accw1_hbm)

    return sc_edges


@functools.lru_cache(maxsize=None)
def _sc_decode_kernel():
    @functools.partial(
        pl.kernel,
        out_type=jax.ShapeDtypeStruct((T,), jnp.float32),
        mesh=_mesh(),
        compiler_params=pltpu.CompilerParams(use_tc_tiling_on_sc=False),
        scratch_types=[
            pltpu.VMEM((QD, KD), jnp.int32),
            pltpu.VMEM((QD, KD), jnp.int32),
            pltpu.VMEM((KD, F2), jnp.float32),
            pltpu.VMEM((KD, F2), jnp.float32),
            pltpu.VMEM((KD, F2), jnp.float32),
            pltpu.VMEM((KD, F2), jnp.float32),
            pltpu.VMEM((KD,), jnp.float32),
            pltpu.VMEM((KD,), jnp.float32),
            pltpu.SemaphoreType.DMA,
            pltpu.SemaphoreType.DMA,
            pltpu.SemaphoreType.DMA,
            pltpu.SemaphoreType.DMA,
        ],
    )
    def sc_decode(z2_hbm, i0p_hbm, i1p_hbm, out_hbm,
                  ibig0, ibig1, za0, za1, zb0, zb1, ob0, ob1,
                  g0, g1, s0, s1):
        c = lax.axis_index("c")
        s = lax.axis_index("s")
        tid = s * NC + c

        za = (za0, za1)
        zb = (zb0, zb1)
        ob = (ob0, ob1)
        gsem = (g0, g1)
        ssem = (s0, s1)

        base_cnt = QD_TOT // NW
        rem = QD_TOT % NW
        cnt = base_cnt + jnp.where(tid < rem, 1, 0)
        start = base_cnt * tid + jnp.minimum(tid, rem)

        pltpu.sync_copy(i0p_hbm.at[pl.ds(start, QD)], ibig0)
        pltpu.sync_copy(i1p_hbm.at[pl.ds(start, QD)], ibig1)

        iot = _iota16()

        def issue_gathers(q, b):
            pltpu.async_copy(z2_hbm.at[ibig0.at[q]], za[b], gsem[b])
            pltpu.async_copy(z2_hbm.at[ibig1.at[q]], zb[b], gsem[b])

        def wait_gathers(b):
            pltpu.make_async_copy(z2_hbm.at[ibig0.at[0]], za[b],
                                  gsem[b]).wait()
            pltpu.make_async_copy(z2_hbm.at[ibig1.at[0]], zb[b],
                                  gsem[b]).wait()

        def compute(b):
            zab = za[b]
            zbb = zb[b]
            obb = ob[b]

            def group_body(g2, carry):
                def edge_body(e, gacc):
                    k = g2 * 16 + e
                    acc = zab[k, pl.ds(0, 16)] * zbb[k, pl.ds(0, 16)]
                    for j in range(1, 4):
                        acc = acc + (zab[k, pl.ds(16 * j, 16)]
                                     * zbb[k, pl.ds(16 * j, 16)])
                    return jnp.where(iot == e, _hsum_all(acc), gacc)

                gacc = lax.fori_loop(0, 16, edge_body,
                                     jnp.zeros((16,), jnp.float32), unroll=4)
                obb[pl.ds(g2 * 16, 16)] = gacc
                return carry

            lax.fori_loop(0, KD // 16, group_body, 0)

        def half(q, b):
            @pl.when(q + 1 < cnt)
            def _():
                issue_gathers(q + 1, 1 - b)

            wait_gathers(b)

            @pl.when(q >= 2)
            def _():
                pltpu.make_async_copy(
                    ob[b], out_hbm.at[pl.ds(0, KD)], ssem[b]).wait()

            compute(b)
            pltpu.async_copy(ob[b], out_hbm.at[pl.ds((start + q) * KD, KD)],
                             ssem[b])

        issue_gathers(0, 0)

        def pair_body(p, carry):
            @pl.when(2 * p < cnt)
            def _():
                half(2 * p, 0)

            @pl.when(2 * p + 1 < cnt)
            def _():
                half(2 * p + 1, 1)

            return carry

        lax.fori_loop(0, QD // 2, pair_body, 0)

        pltpu.make_async_copy(ob[0], out_hbm.at[pl.ds(0, KD)], ssem[0]).wait()
        pltpu.make_async_copy(ob[1], out_hbm.at[pl.ds(0, KD)], ssem[1]).wait()

    return sc_decode


def kernel(x, train_pos_edge_index, test_pos_edge_index, test_neg_edge_index,
           W1, a_src1, a_dst1, b1, W2, a_src2, a_dst2, b2):
    f32 = jnp.float32
    src2d = jnp.zeros((QE_PAD, KE), train_pos_edge_index.dtype).at[
        :QE_TOT].set(train_pos_edge_index[0].reshape(QE_TOT, KE))
    dst2d = jnp.zeros((QE_PAD, KE), train_pos_edge_index.dtype).at[
        :QE_TOT].set(train_pos_edge_index[1].reshape(QE_TOT, KE))
    ei = jnp.concatenate([test_pos_edge_index, test_neg_edge_index], axis=-1)
    npad = QD_TOT * KD // KD
    i0p = jnp.zeros((TPAD // KD, KD), ei.dtype).at[:npad].set(
        ei[0].reshape(npad, KD))
    i1p = jnp.zeros((TPAD // KD, KD), ei.dtype).at[:npad].set(
        ei[1].reshape(npad, KD))

    cols = np.arange(F1)
    perm = (cols % 8) * 8 + cols // 8
    pp = jnp.zeros((F1, F1), f32).at[perm, cols].set(1.0)
    pm = jnp.zeros((F1, F1), f32).at[cols, perm].set(1.0)
    hh = jnp.repeat(jnp.arange(H1), C1)
    asrc_bd = jnp.zeros((F1, 8), f32).at[jnp.arange(F1), hh].set(
        a_src1.reshape(F1))
    adst_bd = jnp.zeros((F1, 8), f32).at[jnp.arange(F1), hh].set(
        a_dst1.reshape(F1))
    p1 = jnp.concatenate([pp, asrc_bd, jnp.zeros((F1, 8), f32)], axis=1)
    ad1m = jnp.concatenate([adst_bd, jnp.zeros((F1, 8), f32)], axis=1)
    b8il = jnp.zeros((8, F1), f32).at[cols % 8, cols].set(1.0)

    p2 = jnp.concatenate(
        [jnp.eye(F2, dtype=f32),
         jnp.outer(a_src2.reshape(F2), jnp.ones((16,), f32))], axis=1)
    ad2m = jnp.outer(a_dst2.reshape(F2), jnp.ones((16,), f32))
    ones64 = jnp.ones((1, F2), f32)

    zr = jnp.zeros((N, R), f32)
    zw = jnp.zeros((N, 16), f32)

    rec1, ad1t = _tc_stage_a(x, W1.T, p1, ad1m)
    a0, a1, a0w, a1w = _sc_edge_kernel(8)(src2d, dst2d, rec1, ad1t, zr, zw)
    rec2, ad2t = _tc_stage_b(a0, a1, a0w, a1w, rec1, ad1t, W2.T, p2, ad2m,
                             b8il, pm, b1.reshape(1, F1))
    b0, b1p, b0w, b1w = _sc_edge_kernel(1)(src2d, dst2d, rec2, ad2t, zr, zw)
    (z2,) = _tc_stage_c(b0, b1p, b0w, b1w, rec2, ad2t, ones64,
                        b2.reshape(1, F2))
    logits = _sc_decode_kernel()(z2, i0p, i1p)
    return logits


# speedup vs baseline: 2.3343x; 1.0260x over previous
import functools

import jax
import jax.numpy as jnp
import numpy as np
from jax import lax
from jax.experimental import pallas as pl
from jax.experimental.pallas import tpu as pltpu
from jax.experimental.pallas import tpu_sc as plsc

N = 10000
D = 128
E = 320000
T = 80000
H1, C1, F1 = 8, 8, 64
F2 = 64

NC, NS = 2, 16
NW = NC * NS
R = 80

KE = 128
QE_TOT = E // KE
QE = QE_TOT // NW + 1
QE_PAD = 2504

KD = 128
QD_TOT = T // KD
QD = 20
TPAD = NW * QD * KD


def _mesh():
    return plsc.VectorSubcoreMesh(
        core_axis_name="c", subcore_axis_name="s",
        num_cores=NC, num_subcores=NS)


def _leaky(x):
    return jnp.maximum(x, 0.2 * x)


def _tca_body(x_ref, w1t_ref, p1_ref, ad1m_ref, rec1_ref, ad1t_ref):
    xp = jnp.dot(x_ref[...], w1t_ref[...], preferred_element_type=jnp.float32)
    rec1_ref[...] = jnp.dot(xp, p1_ref[...], preferred_element_type=jnp.float32)
    ad1t_ref[...] = jnp.dot(xp, ad1m_ref[...],
                            preferred_element_type=jnp.float32)


def _tc_stage_a(x, w1t, p1, ad1m):
    blk = 1000
    return pl.pallas_call(
        _tca_body,
        grid=(N // blk,),
        in_specs=[
            pl.BlockSpec((blk, D), lambda i: (i, 0)),
            pl.BlockSpec((D, F1), lambda i: (0, 0)),
            pl.BlockSpec((F1, R), lambda i: (0, 0)),
            pl.BlockSpec((F1, 16), lambda i: (0, 0)),
        ],
        out_specs=[
            pl.BlockSpec((blk, R), lambda i: (i, 0)),
            pl.BlockSpec((blk, 16), lambda i: (i, 0)),
        ],
        out_shape=[
            jax.ShapeDtypeStruct((N, R), jnp.float32),
            jax.ShapeDtypeStruct((N, 16), jnp.float32),
        ],
    )(x, w1t, p1, ad1m)


def _tcb_body(a0_ref, a1_ref, a0w_ref, a1w_ref, rec1_ref, ad1t_ref, w2t_ref,
              p2_ref, ad2m_ref, b8il_ref, pm_ref, b1_ref, rec2_ref, ad2t_ref):
    xp_il = rec1_ref[:, 0:F1]
    asrc = rec1_ref[:, 64:72]
    adst = ad1t_ref[:, 0:8]
    wself = jnp.exp(_leaky(asrc + adst))
    b8il = b8il_ref[...]
    numer = (a0_ref[:, 0:F1] + a1_ref[:, 0:F1]
             + xp_il * jnp.dot(wself, b8il, preferred_element_type=jnp.float32))
    den8 = a0w_ref[:, 0:8] + a1w_ref[:, 0:8] + wself
    z_il = numer / jnp.dot(den8, b8il, preferred_element_type=jnp.float32)
    z = jnp.dot(z_il, pm_ref[...], preferred_element_type=jnp.float32) \
        + b1_ref[...]
    z = jnp.where(z > 0, z, jnp.exp(z) - 1.0)
    xp2 = jnp.dot(z, w2t_ref[...], preferred_element_type=jnp.float32)
    rec2_ref[...] = jnp.dot(xp2, p2_ref[...],
                            preferred_element_type=jnp.float32)
    ad2t_ref[...] = jnp.dot(xp2, ad2m_ref[...],
                            preferred_element_type=jnp.float32)


def _tc_stage_b(a0, a1, a0w, a1w, rec1, ad1t, w2t, p2, ad2m, b8il, pm, b1row):
    blk = 1000
    return pl.pallas_call(
        _tcb_body,
        grid=(N // blk,),
        in_specs=[
            pl.BlockSpec((blk, R), lambda i: (i, 0)),
            pl.BlockSpec((blk, R), lambda i: (i, 0)),
            pl.BlockSpec((blk, 16), lambda i: (i, 0)),
            pl.BlockSpec((blk, 16), lambda i: (i, 0)),
            pl.BlockSpec((blk, R), lambda i: (i, 0)),
            pl.BlockSpec((blk, 16), lambda i: (i, 0)),
            pl.BlockSpec((F1, F2), lambda i: (0, 0)),
            pl.BlockSpec((F2, R), lambda i: (0, 0)),
            pl.BlockSpec((F2, 16), lambda i: (0, 0)),
            pl.BlockSpec((8, F1), lambda i: (0, 0)),
            pl.BlockSpec((F1, F1), lambda i: (0, 0)),
            pl.BlockSpec((1, F1), lambda i: (0, 0)),
        ],
        out_specs=[
            pl.BlockSpec((blk, R), lambda i: (i, 0)),
            pl.BlockSpec((blk, 16), lambda i: (i, 0)),
        ],
        out_shape=[
            jax.ShapeDtypeStruct((N, R), jnp.float32),
            jax.ShapeDtypeStruct((N, 16), jnp.float32),
        ],
    )(a0, a1, a0w, a1w, rec1, ad1t, w2t, p2, ad2m, b8il, pm, b1row)


def _tcc_body(b0_ref, b1p_ref, b0w_ref, b1w_ref, rec2_ref, ad2t_ref,
              ones64_ref, b2_ref, z2_ref):
    xp2 = rec2_ref[:, 0:F2]
    as2 = rec2_ref[:, 64:65]
    ad2 = ad2t_ref[:, 0:1]
    w2 = jnp.exp(_leaky(as2 + ad2))
    ones64 = ones64_ref[...]
    numer = (b0_ref[:, 0:F2] + b1p_ref[:, 0:F2]
             + xp2 * jnp.dot(w2, ones64, preferred_element_type=jnp.float32))
    denom = jnp.dot(b0w_ref[:, 0:1] + b1w_ref[:, 0:1] + w2, ones64,
                    preferred_element_type=jnp.float32)
    z2_ref[...] = numer / denom + b2_ref[...]


def _tc_stage_c(b0, b1p, b0w, b1w, rec2, ad2t, ones64, b2row):
    blk = 1000
    return pl.pallas_call(
        _tcc_body,
        grid=(N // blk,),
        in_specs=[
            pl.BlockSpec((blk, R), lambda i: (i, 0)),
            pl.BlockSpec((blk, R), lambda i: (i, 0)),
            pl.BlockSpec((blk, 16), lambda i: (i, 0)),
            pl.BlockSpec((blk, 16), lambda i: (i, 0)),
            pl.BlockSpec((blk, R), lambda i: (i, 0)),
            pl.BlockSpec((blk, 16), lambda i: (i, 0)),
            pl.BlockSpec((1, F2), lambda i: (0, 0)),
            pl.BlockSpec((1, F2), lambda i: (0, 0)),
        ],
        out_specs=[pl.BlockSpec((blk, F2), lambda i: (i, 0))],
        out_shape=[jax.ShapeDtypeStruct((N, F2), jnp.float32)],
    )(b0, b1p, b0w, b1w, rec2, ad2t, ones64, b2row)


def _iota16():
    return lax.iota(jnp.int32, 16)


def _lane_gather(v, idx):
    return lax.gather(
        v, idx[:, None],
        dimension_numbers=lax.GatherDimensionNumbers(
            offset_dims=(), collapsed_slice_dims=(0,), start_index_map=(0,)),
        slice_sizes=(1,),
        mode=lax.GatherScatterMode.PROMISE_IN_BOUNDS)


def _hsum_all(v):
    iot = _iota16()
    for off in (8, 4, 2, 1):
        v = v + _lane_gather(v, iot ^ off)
    return v


@functools.lru_cache(maxsize=None)
def _sc_edge_kernel(heads):
    @functools.partial(
        pl.kernel,
        out_type=[
            jax.ShapeDtypeStruct((N, R), jnp.float32),
            jax.ShapeDtypeStruct((N, R), jnp.float32),
            jax.ShapeDtypeStruct((N, 16), jnp.float32),
            jax.ShapeDtypeStruct((N, 16), jnp.float32),
        ],
        mesh=_mesh(),
        compiler_params=pltpu.CompilerParams(use_tc_tiling_on_sc=False),
        scratch_types=[
            pltpu.VMEM((QE, KE), jnp.int32),
            pltpu.VMEM((QE, KE), jnp.int32),

            pltpu.VMEM((KE, R), jnp.float32),
            pltpu.VMEM((KE, R), jnp.float32),
            pltpu.VMEM((KE, 16), jnp.float32),
            pltpu.VMEM((KE, 16), jnp.float32),
            pltpu.VMEM((KE, 16), jnp.float32),
            pltpu.VMEM((KE, 16), jnp.float32),
            pltpu.VMEM_SHARED((N, R), jnp.float32),
            pltpu.VMEM_SHARED((N, 16), jnp.float32),
            pltpu.SemaphoreType.DMA,
            pltpu.SemaphoreType.DMA,
            pltpu.SemaphoreType.DMA,
            pltpu.SemaphoreType.DMA,
        ],
    )
    def sc_edges(src2d_hbm, dst2d_hbm, rec_hbm, adt_hbm, zr_hbm, zw_hbm,
                 accf0_hbm, accf1_hbm, accw0_hbm, accw1_hbm,
                 sbig, dbig, recb0, recb1, adb0, adb1, wb0, wb1,
                 accf_sp, accw_sp, g0, g1, s0, s1):
        c = lax.axis_index("c")
        s = lax.axis_index("s")
        tid = s * NC + c

        recb = (recb0, recb1)
        adb = (adb0, adb1)
        wb = (wb0, wb1)
        gsem = (g0, g1)
        ssem = (s0, s1)

        base_cnt = QE_TOT // NW
        rem = QE_TOT % NW
        cnt = base_cnt + jnp.where(tid < rem, 1, 0)
        start = base_cnt * tid + jnp.minimum(tid, rem)

        pltpu.sync_copy(src2d_hbm.at[pl.ds(start, QE)], sbig)
        pltpu.sync_copy(dst2d_hbm.at[pl.ds(start, QE)], dbig)

        rpt = 624
        tail0 = rpt * NS
        r0 = s * rpt
        pltpu.sync_copy(zr_hbm.at[pl.ds(r0, rpt)], accf_sp.at[pl.ds(r0, rpt)])
        pltpu.sync_copy(zw_hbm.at[pl.ds(r0, rpt)], accw_sp.at[pl.ds(r0, rpt)])

        @pl.when(s == NS - 1)
        def _():
            pltpu.sync_copy(zr_hbm.at[pl.ds(tail0, N - tail0)],
                            accf_sp.at[pl.ds(tail0, N - tail0)])
            pltpu.sync_copy(zw_hbm.at[pl.ds(tail0, N - tail0)],
                            accw_sp.at[pl.ds(tail0, N - tail0)])

        plsc.subcore_barrier()

        iot = _iota16()
        pat8 = iot & 7

        def issue_gathers(q, b):
            pltpu.async_copy(rec_hbm.at[sbig.at[q]], recb[b], gsem[b])
            pltpu.async_copy(adt_hbm.at[dbig.at[q]], adb[b], gsem[b])

        def wait_gathers(b):
            pltpu.make_async_copy(rec_hbm.at[sbig.at[0]], recb[b],
                                  gsem[b]).wait()
            pltpu.make_async_copy(adt_hbm.at[dbig.at[0]], adb[b],
                                  gsem[b]).wait()

        def issue_scatters(q, b):
            pltpu.async_copy(recb[b], accf_sp.at[dbig.at[q]], ssem[b],
                             add=True)
            pltpu.async_copy(wb[b], accw_sp.at[dbig.at[q]], ssem[b], add=True)

        def wait_scatters(b):
            pltpu.make_async_copy(recb[b], accf_sp.at[dbig.at[0]],
                                  ssem[b]).wait()
            pltpu.make_async_copy(wb[b], accw_sp.at[dbig.at[0]],
                                  ssem[b]).wait()

        def compute(b):
            rb = recb[b]
            ab = adb[b]
            wbb = wb[b]

            @plsc.parallel_loop(0, KE, unroll=4)
            def edge_body(k):
                alpha = rb[k, pl.ds(64, 16)] + ab[k, :]
                w = jnp.exp(_leaky(alpha))
                wbb[k, :] = w
                wv = _lane_gather(w, pat8) if heads == 8 else w
                for j in range(4):
                    rb[k, pl.ds(16 * j, 16)] = rb[k, pl.ds(16 * j, 16)] * wv

        def half(q, b):
            @pl.when(q >= 1)
            def _():
                wait_scatters(1 - b)

            @pl.when(q + 1 < cnt)
            def _():
                issue_gathers(q + 1, 1 - b)

            wait_gathers(b)
            compute(b)
            issue_scatters(q, b)

        issue_gathers(0, 0)

        def pair_body(p, carry):
            @pl.when(2 * p < cnt)
            def _():
                half(2 * p, 0)

            @pl.when(2 * p + 1 < cnt)
            def _():
                half(2 * p + 1, 1)

            return carry

        lax.fori_loop(0, QE // 2 + 1, pair_body, 0)
        @pl.when(cnt % 2 == 1)
        def _():
            wait_scatters(0)

        @pl.when(cnt % 2 == 0)
        def _():
            wait_scatters(1)

        plsc.subcore_barrier()

        def dump(sp_ref, hbm_ref):
            pltpu.sync_copy(sp_ref.at[pl.ds(r0, rpt)],
                            hbm_ref.at[pl.ds(r0, rpt)])

            @pl.when(s == NS - 1)
            def _():
                pltpu.sync_copy(sp_ref.at[pl.ds(tail0, N - tail0)],
                                hbm_ref.at[pl.ds(tail0, N - tail0)])

        @pl.when(c == 0)
        def _():
            dump(accf_sp, accf0_hbm)
            dump(accw_sp, accw0_hbm)

        @pl.when(c == 1)
        def _():
            dump(accf_sp, accf1_hbm)
            dump(accw_sp, accw1_hbm)

    return sc_edges


@functools.lru_cache(maxsize=None)
def _sc_decode_kernel():
    RC = 125

    @functools.partial(
        pl.kernel,
        out_type=jax.ShapeDtypeStruct((T,), jnp.float32),
        mesh=_mesh(),
        compiler_params=pltpu.CompilerParams(use_tc_tiling_on_sc=False),
        scratch_types=[
            pltpu.VMEM((QD, KD), jnp.int32),
            pltpu.VMEM((QD, KD), jnp.int32),
            pltpu.VMEM((KD, F2), jnp.float32),
            pltpu.VMEM((KD, F2), jnp.float32),
            pltpu.VMEM((KD, F2), jnp.float32),
            pltpu.VMEM((KD, F2), jnp.float32),
            pltpu.VMEM((KD,), jnp.float32),
            pltpu.VMEM((KD,), jnp.float32),
            pltpu.VMEM((RC, R), jnp.float32),
            pltpu.VMEM((RC, R), jnp.float32),
            pltpu.VMEM((RC, R), jnp.float32),
            pltpu.VMEM((RC, 16), jnp.float32),
            pltpu.VMEM((RC, 16), jnp.float32),
            pltpu.VMEM((RC, 16), jnp.float32),
            pltpu.VMEM((RC, F2), jnp.float32),
            pltpu.VMEM((F2,), jnp.float32),
            pltpu.VMEM_SHARED((N, F2), jnp.float32),
            pltpu.SemaphoreType.DMA,
            pltpu.SemaphoreType.DMA,
            pltpu.SemaphoreType.DMA,
            pltpu.SemaphoreType.DMA,
        ],
    )
    def sc_decode(b0_hbm, b1p_hbm, b0w_hbm, b1w_hbm, rec2_hbm, ad2t_hbm,
                  b2_hbm, i0p_hbm, i1p_hbm, out_hbm,
                  ibig0, ibig1, za0, za1, zb0, zb1, ob0, ob1,
                  m0, m1, mr, ma, mw0, mw1, mz, bb, z2_sp,
                  g0, g1, s0, s1):
        c = lax.axis_index("c")
        s = lax.axis_index("s")
        tid = s * NC + c

        za = (za0, za1)
        zb = (zb0, zb1)
        ob = (ob0, ob1)
        gsem = (g0, g1)
        ssem = (s0, s1)

        pltpu.sync_copy(b2_hbm, bb)

        def merge_chunk(i, carry):
            r0 = s * (RC * 5) + i * RC
            pltpu.sync_copy(b0_hbm.at[pl.ds(r0, RC)], m0)
            pltpu.sync_copy(b1p_hbm.at[pl.ds(r0, RC)], m1)
            pltpu.sync_copy(rec2_hbm.at[pl.ds(r0, RC)], mr)
            pltpu.sync_copy(ad2t_hbm.at[pl.ds(r0, RC)], ma)
            pltpu.sync_copy(b0w_hbm.at[pl.ds(r0, RC)], mw0)
            pltpu.sync_copy(b1w_hbm.at[pl.ds(r0, RC)], mw1)

            @plsc.parallel_loop(0, RC, unroll=4)
            def row_body(k):
                alpha = mr[k, pl.ds(64, 16)] + ma[k, :]
                w = jnp.exp(_leaky(alpha))
                den = mw0[k, :] + mw1[k, :] + w
                for j in range(4):
                    numer = (m0[k, pl.ds(16 * j, 16)]
                             + m1[k, pl.ds(16 * j, 16)]
                             + mr[k, pl.ds(16 * j, 16)] * w)
                    mz[k, pl.ds(16 * j, 16)] = (
                        numer / den + bb[pl.ds(16 * j, 16)])

            pltpu.sync_copy(mz, z2_sp.at[pl.ds(r0, RC)])
            return carry

        lax.fori_loop(0, 5, merge_chunk, 0)
        plsc.subcore_barrier()

        base_cnt = QD_TOT // NW
        rem = QD_TOT % NW
        cnt = base_cnt + jnp.where(tid < rem, 1, 0)
        start = base_cnt * tid + jnp.minimum(tid, rem)

        pltpu.sync_copy(i0p_hbm.at[pl.ds(start, QD)], ibig0)
        pltpu.sync_copy(i1p_hbm.at[pl.ds(start, QD)], ibig1)

        iot = _iota16()

        def issue_gathers(q, b):
            pltpu.async_copy(z2_sp.at[ibig0.at[q]], za[b], gsem[b])
            pltpu.async_copy(z2_sp.at[ibig1.at[q]], zb[b], gsem[b])

        def wait_gathers(b):
            pltpu.make_async_copy(z2_sp.at[ibig0.at[0]], za[b],
                                  gsem[b]).wait()
            pltpu.make_async_copy(z2_sp.at[ibig1.at[0]], zb[b],
                                  gsem[b]).wait()

        def compute(b):
            zab = za[b]
            zbb = zb[b]
            obb = ob[b]

            def group_body(g2, carry):
                def edge_body(e, gacc):
                    k = g2 * 16 + e
                    acc = zab[k, pl.ds(0, 16)] * zbb[k, pl.ds(0, 16)]
                    for j in range(1, 4):
                        acc = acc + (zab[k, pl.ds(16 * j, 16)]
                                     * zbb[k, pl.ds(16 * j, 16)])
                    return jnp.where(iot == e, _hsum_all(acc), gacc)

                gacc = lax.fori_loop(0, 16, edge_body,
                                     jnp.zeros((16,), jnp.float32), unroll=4)
                obb[pl.ds(g2 * 16, 16)] = gacc
                return carry

            lax.fori_loop(0, KD // 16, group_body, 0)

        def half(q, b):
            @pl.when(q + 1 < cnt)
            def _():
                issue_gathers(q + 1, 1 - b)

            wait_gathers(b)

            @pl.when(q >= 2)
            def _():
                pltpu.make_async_copy(
                    ob[b], out_hbm.at[pl.ds(0, KD)], ssem[b]).wait()

            compute(b)
            pltpu.async_copy(ob[b], out_hbm.at[pl.ds((start + q) * KD, KD)],
                             ssem[b])

        issue_gathers(0, 0)

        def pair_body(p, carry):
            @pl.when(2 * p < cnt)
            def _():
                half(2 * p, 0)

            @pl.when(2 * p + 1 < cnt)
            def _():
                half(2 * p + 1, 1)

            return carry

        lax.fori_loop(0, QD // 2, pair_body, 0)

        pltpu.make_async_copy(ob[0], out_hbm.at[pl.ds(0, KD)], ssem[0]).wait()
        pltpu.make_async_copy(ob[1], out_hbm.at[pl.ds(0, KD)], ssem[1]).wait()

    return sc_decode


def kernel(x, train_pos_edge_index, test_pos_edge_index, test_neg_edge_index,
           W1, a_src1, a_dst1, b1, W2, a_src2, a_dst2, b2):
    f32 = jnp.float32
    src2d = jnp.zeros((QE_PAD, KE), train_pos_edge_index.dtype).at[
        :QE_TOT].set(train_pos_edge_index[0].reshape(QE_TOT, KE))
    dst2d = jnp.zeros((QE_PAD, KE), train_pos_edge_index.dtype).at[
        :QE_TOT].set(train_pos_edge_index[1].reshape(QE_TOT, KE))
    ei = jnp.concatenate([test_pos_edge_index, test_neg_edge_index], axis=-1)
    npad = QD_TOT * KD // KD
    i0p = jnp.zeros((TPAD // KD, KD), ei.dtype).at[:npad].set(
        ei[0].reshape(npad, KD))
    i1p = jnp.zeros((TPAD // KD, KD), ei.dtype).at[:npad].set(
        ei[1].reshape(npad, KD))

    cols = np.arange(F1)
    perm = (cols % 8) * 8 + cols // 8
    pp = jnp.zeros((F1, F1), f32).at[perm, cols].set(1.0)
    pm = jnp.zeros((F1, F1), f32).at[cols, perm].set(1.0)
    hh = jnp.repeat(jnp.arange(H1), C1)
    asrc_bd = jnp.zeros((F1, 8), f32).at[jnp.arange(F1), hh].set(
        a_src1.reshape(F1))
    adst_bd = jnp.zeros((F1, 8), f32).at[jnp.arange(F1), hh].set(
        a_dst1.reshape(F1))
    p1 = jnp.concatenate([pp, asrc_bd, jnp.zeros((F1, 8), f32)], axis=1)
    ad1m = jnp.concatenate([adst_bd, jnp.zeros((F1, 8), f32)], axis=1)
    b8il = jnp.zeros((8, F1), f32).at[cols % 8, cols].set(1.0)

    p2 = jnp.concatenate(
        [jnp.eye(F2, dtype=f32),
         jnp.outer(a_src2.reshape(F2), jnp.ones((16,), f32))], axis=1)
    ad2m = jnp.outer(a_dst2.reshape(F2), jnp.ones((16,), f32))
    ones64 = jnp.ones((1, F2), f32)

    zr = jnp.zeros((N, R), f32)
    zw = jnp.zeros((N, 16), f32)

    rec1, ad1t = _tc_stage_a(x, W1.T, p1, ad1m)
    a0, a1, a0w, a1w = _sc_edge_kernel(8)(src2d, dst2d, rec1, ad1t, zr, zw)
    rec2, ad2t = _tc_stage_b(a0, a1, a0w, a1w, rec1, ad1t, W2.T, p2, ad2m,
                             b8il, pm, b1.reshape(1, F1))
    b0, b1p, b0w, b1w = _sc_edge_kernel(1)(src2d, dst2d, rec2, ad2t, zr, zw)
    logits = _sc_decode_kernel()(b0, b1p, b0w, b1w, rec2, ad2t, b2, i0p, i1p)
    return logits
